# Initial kernel scaffold; baseline (speedup 1.0000x reference)
import jax
import jax.numpy as jnp
from jax.experimental import pallas as pl


def kernel(constraint_features_s, edge_index_s, edge_attr_s, variable_features_s, bounds_s, constraint_features_t, edge_index_t, edge_attr_t, variable_features_t, bounds_t, params):
    raise NotImplementedError("write your pallas kernel here")


# speedup vs baseline: 25.6084x; 25.6084x over previous
import functools

import jax
import jax.numpy as jnp
from jax import lax
from jax.experimental import pallas as pl
from jax.experimental.pallas import tpu as pltpu
from jax.experimental.pallas import tpu_sc as plsc

N_NODES = 50000
N_PAD = 50048
DUMMY = N_NODES
NC, NS = 2, 16
NW = NC * NS
EB = 128
RPC = 4
BLK = N_PAD // 8
MBLK = N_PAD // 16


def _edge_pad(e):
    per = NW * RPC * EB
    return ((e + per - 1) // per) * per


@functools.lru_cache(maxsize=None)
def _sc_pass(d, e_pad):
    rows_total = e_pad // EB
    rows_pw = rows_total // NW
    chunks = rows_pw // RPC
    out_rows = N_PAD // NS
    mesh = plsc.VectorSubcoreMesh(core_axis_name="c", subcore_axis_name="s")

    def body(idx0_hbm, idx1_hbm, tabA_hbm, tabB_hbm, zeros_hbm,
             aggA_hbm, aggB_hbm,
             idx0_v, idx1_v, rowsA_v, rowsB_v, accA_sh, accB_sh,
             sem_i, sem_g, sem_s):
        c = lax.axis_index("c")
        s = lax.axis_index("s")
        w = c * NS + s

        @pl.when(s == 0)
        def _init():
            pltpu.sync_copy(zeros_hbm, accA_sh)
            pltpu.sync_copy(zeros_hbm, accB_sh)

        plsc.subcore_barrier()

        def chunk_body(t, carry):
            row0 = w * rows_pw + t * RPC
            cp0 = pltpu.async_copy(idx0_hbm.at[pl.ds(row0, RPC)], idx0_v, sem_i)
            cp1 = pltpu.async_copy(idx1_hbm.at[pl.ds(row0, RPC)], idx1_v, sem_i)
            cp0.wait()
            cp1.wait()
            gs = []
            for j in range(RPC):
                gs.append(pltpu.async_copy(
                    tabA_hbm.at[idx0_v.at[j]],
                    rowsA_v.at[pl.ds(j * EB, EB)], sem_g))
                gs.append(pltpu.async_copy(
                    tabB_hbm.at[idx1_v.at[j]],
                    rowsB_v.at[pl.ds(j * EB, EB)], sem_g))
            for cp in gs:
                cp.wait()
            ss = []
            for j in range(RPC):
                ss.append(pltpu.async_copy(
                    rowsA_v.at[pl.ds(j * EB, EB)],
                    accA_sh.at[idx1_v.at[j]], sem_s, add=True))
                ss.append(pltpu.async_copy(
                    rowsB_v.at[pl.ds(j * EB, EB)],
                    accB_sh.at[idx0_v.at[j]], sem_s, add=True))
            for cp in ss:
                cp.wait()
            return carry

        lax.fori_loop(0, chunks, chunk_body, 0)
        plsc.subcore_barrier()
        out0 = c * N_PAD + s * out_rows
        pltpu.sync_copy(accA_sh.at[pl.ds(s * out_rows, out_rows)],
                        aggA_hbm.at[pl.ds(out0, out_rows)])
        pltpu.sync_copy(accB_sh.at[pl.ds(s * out_rows, out_rows)],
                        aggB_hbm.at[pl.ds(out0, out_rows)])

    return pl.kernel(
        body,
        out_type=(jax.ShapeDtypeStruct((2 * N_PAD, d), jnp.float32),
                  jax.ShapeDtypeStruct((2 * N_PAD, d), jnp.float32)),
        mesh=mesh,
        scratch_types=[
            pltpu.VMEM((RPC, EB), jnp.int32),
            pltpu.VMEM((RPC, EB), jnp.int32),
            pltpu.VMEM((RPC * EB, d), jnp.float32),
            pltpu.VMEM((RPC * EB, d), jnp.float32),
            pltpu.VMEM_SHARED((N_PAD, d), jnp.float32),
            pltpu.VMEM_SHARED((N_PAD, d), jnp.float32),
            pltpu.SemaphoreType.DMA,
            pltpu.SemaphoreType.DMA,
            pltpu.SemaphoreType.DMA,
        ],
        compiler_params=pltpu.CompilerParams(use_tc_tiling_on_sc=False),
    )


_DOT = functools.partial(jnp.dot, precision=lax.Precision.HIGHEST)


@functools.lru_cache(maxsize=None)
def _tc_stage0():
    def body(vf, g6, b6, WvT, bv, WrootT, cWrelT, rootv_o, tabv_o):
        x = vf[...]
        m = jnp.mean(x, -1, keepdims=True)
        v = jnp.mean((x - m) ** 2, -1, keepdims=True)
        xn = (x - m) * lax.rsqrt(v + 1e-5) * g6[...] + b6[...]
        var0 = jax.nn.relu(_DOT(xn, WvT[...]) + bv[...])
        rootv_o[...] = _DOT(var0, WrootT[...])
        tabv_o[...] = _DOT(var0, cWrelT[...])

    def full(shape):
        return pl.BlockSpec(shape, lambda i: (0, 0))

    return pl.pallas_call(
        body,
        grid=(8,),
        in_specs=[pl.BlockSpec((BLK, 6), lambda i: (i, 0)),
                  full((1, 6)), full((1, 6)), full((6, 32)), full((1, 32)),
                  full((32, 8)), full((32, 8))],
        out_specs=[pl.BlockSpec((BLK, 8), lambda i: (i, 0)),
                   pl.BlockSpec((BLK, 8), lambda i: (i, 0))],
        out_shape=[jax.ShapeDtypeStruct((N_PAD, 8), jnp.float32),
                   jax.ShapeDtypeStruct((N_PAD, 8), jnp.float32)],
    )


@functools.lru_cache(maxsize=None)
def _tc_stage_mid(d_in, d_out, rootc_bcast):
    def body(a0, a1, b0, b1, rootc, rootv, brel, WrootT, cWrelT,
             rootc_o, rootv_o, tabc_o, tabv_o):
        consr = jax.nn.relu(a0[...] + a1[...] + brel[...] + rootc[...])
        varr = jax.nn.relu(b0[...] + b1[...] + brel[...] + rootv[...])
        rootc_o[...] = _DOT(consr, WrootT[...])
        rootv_o[...] = _DOT(varr, WrootT[...])
        tabc_o[...] = _DOT(consr, cWrelT[...])
        tabv_o[...] = _DOT(varr, cWrelT[...])

    def full(shape):
        return pl.BlockSpec(shape, lambda i: (0, 0))

    rootc_spec = (full((1, d_in)) if rootc_bcast
                  else pl.BlockSpec((MBLK, d_in), lambda i: (i, 0)))
    return pl.pallas_call(
        body,
        grid=(16,),
        in_specs=[pl.BlockSpec((MBLK, d_in), lambda i: (i, 0)),
                  pl.BlockSpec((MBLK, d_in), lambda i: (i + 16, 0)),
                  pl.BlockSpec((MBLK, d_in), lambda i: (i, 0)),
                  pl.BlockSpec((MBLK, d_in), lambda i: (i + 16, 0)),
                  rootc_spec,
                  pl.BlockSpec((MBLK, d_in), lambda i: (i, 0)),
                  full((1, d_in)), full((d_in, d_out)), full((d_in, d_out))],
        out_specs=[pl.BlockSpec((MBLK, d_out), lambda i: (i, 0))] * 4,
        out_shape=[jax.ShapeDtypeStruct((N_PAD, d_out), jnp.float32)] * 4,
    )


@functools.lru_cache(maxsize=None)
def _tc_stage3():
    def body(a0, a1, b0, b1, rootc, rootv, brel, out):
        i = pl.program_id(0)
        gid = i * MBLK + lax.broadcasted_iota(jnp.int32, (MBLK, 1), 0)
        mask = (gid < N_NODES).astype(jnp.float32)
        consr = jax.nn.relu(a0[...] + a1[...] + brel[...] + rootc[...]) * mask
        varr = jax.nn.relu(b0[...] + b1[...] + brel[...] + rootv[...]) * mask
        part = jnp.concatenate([jnp.sum(consr, 0, keepdims=True),
                                jnp.sum(varr, 0, keepdims=True)], axis=0)

        @pl.when(i == 0)
        def _zero():
            out[...] = jnp.zeros_like(out)

        out[...] += part

    def full(shape):
        return pl.BlockSpec(shape, lambda i: (0, 0))

    return pl.pallas_call(
        body,
        grid=(16,),
        in_specs=[pl.BlockSpec((MBLK, 4), lambda i: (i, 0)),
                  pl.BlockSpec((MBLK, 4), lambda i: (i + 16, 0)),
                  pl.BlockSpec((MBLK, 4), lambda i: (i, 0)),
                  pl.BlockSpec((MBLK, 4), lambda i: (i + 16, 0)),
                  pl.BlockSpec((MBLK, 4), lambda i: (i, 0)),
                  pl.BlockSpec((MBLK, 4), lambda i: (i, 0)),
                  full((1, 4))],
        out_specs=full((2, 4)),
        out_shape=jax.ShapeDtypeStruct((2, 4), jnp.float32),
    )


def _ln_row(x, g, b, eps=1e-5):
    m = jnp.mean(x, -1, keepdims=True)
    v = jnp.var(x, -1, keepdims=True)
    return (x - m) / jnp.sqrt(v + eps) * g + b


def kernel(constraint_features_s, edge_index_s, edge_attr_s,
           variable_features_s, bounds_s,
           constraint_features_t, edge_index_t, edge_attr_t,
           variable_features_t, bounds_t, params):
    p = params
    relu = jax.nn.relu
    e_pad = _edge_pad(edge_index_s.shape[1])

    c = p['ln_edge_b'][0]
    cons0row = relu(p['ln_cons_b'][0] * p['W_cons'][:, 0] + p['b_cons'])
    rootc1 = (cons0row @ p['Wroot1'].T).reshape(1, 8)
    tabC1row = (c * (cons0row @ p['Wrel1'].T)).reshape(1, 8)
    g6 = p['ln_var_g'].reshape(1, 6)
    b6 = p['ln_var_b'].reshape(1, 6)
    WvT = p['W_var'].T
    bv = p['b_var'].reshape(1, 32)
    weights = {
        1: (p['brel1'].reshape(1, 8), p['Wroot1'].T, c * p['Wrel1'].T),
        2: (p['brel2'].reshape(1, 4), p['Wroot2'].T, c * p['Wrel2'].T),
        3: (p['brel3'].reshape(1, 4), p['Wroot3'].T, c * p['Wrel3'].T),
    }
    zeros8 = jnp.zeros((N_PAD, 8), jnp.float32)
    zeros4 = jnp.zeros((N_PAD, 4), jnp.float32)
    tabC1 = jnp.tile(tabC1row, (N_PAD, 1))

    stage0 = _tc_stage0()
    mid1 = _tc_stage_mid(8, 4, True)
    mid2 = _tc_stage_mid(4, 4, False)
    stage3 = _tc_stage3()
    pass8 = _sc_pass(8, e_pad)
    pass4 = _sc_pass(4, e_pad)

    def run_graph(edge_index, var_f, bounds):
        e = edge_index.shape[1]
        padv = jnp.full((e_pad - e,), DUMMY, jnp.int32)
        idx0 = jnp.concatenate([edge_index[0].astype(jnp.int32), padv]
                               ).reshape(-1, EB)
        idx1 = jnp.concatenate([edge_index[1].astype(jnp.int32), padv]
                               ).reshape(-1, EB)
        vfp = jnp.pad(var_f, ((0, N_PAD - var_f.shape[0]), (0, 0)))

        rootv1, tabV1 = stage0(vfp, g6, b6, WvT, bv, weights[1][1],
                               weights[1][2])
        aggA, aggB = pass8(idx0, idx1, tabV1, tabC1, zeros8)
        rootc2, rootv2, tabC2, tabV2 = mid1(
            aggA, aggA, aggB, aggB, rootc1, rootv1, weights[1][0],
            weights[2][1], weights[2][2])
        aggA, aggB = pass4(idx0, idx1, tabV2, tabC2, zeros4)
        rootc3, rootv3, tabC3, tabV3 = mid2(
            aggA, aggA, aggB, aggB, rootc2, rootv2, weights[2][0],
            weights[3][1], weights[3][2])
        aggA, aggB = pass4(idx0, idx1, tabV3, tabC3, zeros4)
        sums = stage3(aggA, aggA, aggB, aggB, rootc3, rootv3, weights[3][0])

        bnd = relu(_ln_row(bounds, p['ln_bnd_g'], p['ln_bnd_b'])
                   @ p['W_bnd'].T + p['b_bnd'])
        cons_avg = sums[0:1] / N_NODES
        var_avg = sums[1:2] / N_NODES
        return jnp.concatenate([var_avg, cons_avg, bnd], axis=1)

    out0 = run_graph(edge_index_s, variable_features_s, bounds_s)
    out1 = run_graph(edge_index_t, variable_features_t, bounds_t)
    score0 = jnp.linalg.norm(out0, axis=1)
    score1 = jnp.linalg.norm(out1, axis=1)
    return jax.nn.sigmoid(-score0 + score1)


# speedup vs baseline: 29.7714x; 1.1626x over previous
import functools

import jax
import jax.numpy as jnp
from jax import lax
from jax.experimental import pallas as pl
from jax.experimental.pallas import tpu as pltpu
from jax.experimental.pallas import tpu_sc as plsc

N_NODES = 50000
N_PAD = 50048
DUMMY = N_NODES
NC, NS = 2, 16
EB = 128
RPC = 4
BLK = N_PAD // 8
MBLK = N_PAD // 16
OUT_ROWS = N_PAD // NS


def _edge_pad(e):
    per = NS * RPC * EB
    return ((e + per - 1) // per) * per


@functools.lru_cache(maxsize=None)
def _sc_pass(d, e_pad):
    rows_pg = e_pad // EB
    rows_pw = rows_pg // NS
    n_chunks = rows_pw // RPC
    mesh = plsc.VectorSubcoreMesh(core_axis_name="c", subcore_axis_name="s")

    def body(idx0_hbm, idx1_hbm, tabA_hbm, tabB_hbm, zeros_hbm,
             aggA_hbm, aggB_hbm,
             idx0_v, idx1_v, rowsA_v, rowsB_v,
             tabA_sh, tabB_sh, accA_sh, accB_sh,
             sem_i, sem_g, sem_s):
        c = lax.axis_index("c")
        s = lax.axis_index("s")
        src0 = c * N_PAD + s * OUT_ROWS
        dst0 = s * OUT_ROWS
        pltpu.sync_copy(tabA_hbm.at[pl.ds(src0, OUT_ROWS)],
                        tabA_sh.at[pl.ds(dst0, OUT_ROWS)])
        pltpu.sync_copy(tabB_hbm.at[pl.ds(src0, OUT_ROWS)],
                        tabB_sh.at[pl.ds(dst0, OUT_ROWS)])
        pltpu.sync_copy(zeros_hbm.at[pl.ds(dst0, OUT_ROWS)],
                        accA_sh.at[pl.ds(dst0, OUT_ROWS)])
        pltpu.sync_copy(zeros_hbm.at[pl.ds(dst0, OUT_ROWS)],
                        accB_sh.at[pl.ds(dst0, OUT_ROWS)])
        plsc.subcore_barrier()

        row_base = c * rows_pg + s * rows_pw

        def issue_idx(t, b):
            cp0 = pltpu.async_copy(
                idx0_hbm.at[pl.ds(row_base + t * RPC, RPC)],
                idx0_v.at[b], sem_i)
            cp1 = pltpu.async_copy(
                idx1_hbm.at[pl.ds(row_base + t * RPC, RPC)],
                idx1_v.at[b], sem_i)
            return cp0, cp1

        def wait_idx(b):
            pltpu.make_async_copy(idx0_hbm.at[pl.ds(0, RPC)],
                                  idx0_v.at[b], sem_i).wait()
            pltpu.make_async_copy(idx1_hbm.at[pl.ds(0, RPC)],
                                  idx1_v.at[b], sem_i).wait()

        def issue_gather(b):
            for j in range(RPC):
                pltpu.async_copy(tabA_sh.at[idx0_v.at[b, j]],
                                 rowsA_v.at[b, pl.ds(j * EB, EB)], sem_g)
                pltpu.async_copy(tabB_sh.at[idx1_v.at[b, j]],
                                 rowsB_v.at[b, pl.ds(j * EB, EB)], sem_g)

        def wait_gather(b):
            for j in range(RPC):
                pltpu.make_async_copy(tabA_sh.at[idx0_v.at[b, j]],
                                      rowsA_v.at[b, pl.ds(j * EB, EB)],
                                      sem_g).wait()
                pltpu.make_async_copy(tabB_sh.at[idx1_v.at[b, j]],
                                      rowsB_v.at[b, pl.ds(j * EB, EB)],
                                      sem_g).wait()

        def issue_scatter(b):
            for j in range(RPC):
                pltpu.async_copy(rowsA_v.at[b, pl.ds(j * EB, EB)],
                                 accA_sh.at[idx1_v.at[b, j]], sem_s, add=True)
                pltpu.async_copy(rowsB_v.at[b, pl.ds(j * EB, EB)],
                                 accB_sh.at[idx0_v.at[b, j]], sem_s, add=True)

        def wait_scatter(b):
            for j in range(RPC):
                pltpu.make_async_copy(rowsA_v.at[b, pl.ds(j * EB, EB)],
                                      accA_sh.at[idx1_v.at[b, j]],
                                      sem_s).wait()
                pltpu.make_async_copy(rowsB_v.at[b, pl.ds(j * EB, EB)],
                                      accB_sh.at[idx0_v.at[b, j]],
                                      sem_s).wait()

        issue_idx(0, 0)
        issue_idx(1, 1)
        wait_idx(0)
        issue_gather(0)

        def chunk_body(t, carry):
            cur = lax.rem(t, 2)
            nxt = 1 - cur

            def on(b):
                wait_gather(b)
                issue_scatter(b)

                @pl.when(t + 1 < n_chunks)
                def _g():
                    wait_idx(1 - b)
                    issue_gather(1 - b)

                wait_scatter(b)

                @pl.when(t + 2 < n_chunks)
                def _i():
                    issue_idx(t + 2, b)

            @pl.when(cur == 0)
            def _b0():
                on(0)

            @pl.when(cur == 1)
            def _b1():
                on(1)

            return carry

        lax.fori_loop(0, n_chunks, chunk_body, 0)
        plsc.subcore_barrier()
        out0 = c * N_PAD + s * OUT_ROWS
        pltpu.sync_copy(accA_sh.at[pl.ds(dst0, OUT_ROWS)],
                        aggA_hbm.at[pl.ds(out0, OUT_ROWS)])
        pltpu.sync_copy(accB_sh.at[pl.ds(dst0, OUT_ROWS)],
                        aggB_hbm.at[pl.ds(out0, OUT_ROWS)])

    return pl.kernel(
        body,
        out_type=(jax.ShapeDtypeStruct((2 * N_PAD, d), jnp.float32),
                  jax.ShapeDtypeStruct((2 * N_PAD, d), jnp.float32)),
        mesh=mesh,
        scratch_types=[
            pltpu.VMEM((2, RPC, EB), jnp.int32),
            pltpu.VMEM((2, RPC, EB), jnp.int32),
            pltpu.VMEM((2, RPC * EB, d), jnp.float32),
            pltpu.VMEM((2, RPC * EB, d), jnp.float32),
            pltpu.VMEM_SHARED((N_PAD, d), jnp.float32),
            pltpu.VMEM_SHARED((N_PAD, d), jnp.float32),
            pltpu.VMEM_SHARED((N_PAD, d), jnp.float32),
            pltpu.VMEM_SHARED((N_PAD, d), jnp.float32),
            pltpu.SemaphoreType.DMA,
            pltpu.SemaphoreType.DMA,
            pltpu.SemaphoreType.DMA,
        ],
        compiler_params=pltpu.CompilerParams(use_tc_tiling_on_sc=False),
    )


_DOT = functools.partial(jnp.dot, precision=lax.Precision.HIGHEST)


@functools.lru_cache(maxsize=None)
def _tc_stage0():
    def body(vf, g6, b6, WvT, bv, WrootT, cWrelT, rootv_o, tabv_o):
        x = vf[...]
        m = jnp.mean(x, -1, keepdims=True)
        v = jnp.mean((x - m) ** 2, -1, keepdims=True)
        xn = (x - m) * lax.rsqrt(v + 1e-5) * g6[...] + b6[...]
        var0 = jax.nn.relu(_DOT(xn, WvT[...]) + bv[...])
        rootv_o[...] = _DOT(var0, WrootT[...])
        tabv_o[...] = _DOT(var0, cWrelT[...])

    def full(shape):
        return pl.BlockSpec(shape, lambda i: (0, 0))

    return pl.pallas_call(
        body,
        grid=(16,),
        in_specs=[pl.BlockSpec((BLK, 6), lambda i: (i, 0)),
                  full((1, 6)), full((1, 6)), full((6, 32)), full((1, 32)),
                  full((32, 8)), full((32, 8))],
        out_specs=[pl.BlockSpec((BLK, 8), lambda i: (i, 0)),
                   pl.BlockSpec((BLK, 8), lambda i: (i, 0))],
        out_shape=[jax.ShapeDtypeStruct((2 * N_PAD, 8), jnp.float32),
                   jax.ShapeDtypeStruct((2 * N_PAD, 8), jnp.float32)],
    )


@functools.lru_cache(maxsize=None)
def _tc_stage_mid(d_in, d_out, rootc_bcast):
    def body(a0, b0, rootc, rootv, brel, WrootT, cWrelT,
             rootc_o, rootv_o, tabc_o, tabv_o):
        consr = jax.nn.relu(a0[...] + brel[...] + rootc[...])
        varr = jax.nn.relu(b0[...] + brel[...] + rootv[...])
        rootc_o[...] = _DOT(consr, WrootT[...])
        rootv_o[...] = _DOT(varr, WrootT[...])
        tabc_o[...] = _DOT(consr, cWrelT[...])
        tabv_o[...] = _DOT(varr, cWrelT[...])

    def full(shape):
        return pl.BlockSpec(shape, lambda i: (0, 0))

    rootc_spec = (full((1, d_in)) if rootc_bcast
                  else pl.BlockSpec((MBLK, d_in), lambda i: (i, 0)))
    return pl.pallas_call(
        body,
        grid=(32,),
        in_specs=[pl.BlockSpec((MBLK, d_in), lambda i: (i, 0)),
                  pl.BlockSpec((MBLK, d_in), lambda i: (i, 0)),
                  rootc_spec,
                  pl.BlockSpec((MBLK, d_in), lambda i: (i, 0)),
                  full((1, d_in)), full((d_in, d_out)), full((d_in, d_out))],
        out_specs=[pl.BlockSpec((MBLK, d_out), lambda i: (i, 0))] * 4,
        out_shape=[jax.ShapeDtypeStruct((2 * N_PAD, d_out), jnp.float32)] * 4,
    )


@functools.lru_cache(maxsize=None)
def _tc_stage3():
    def body(a0, b0, rootc, rootv, brel, out):
        i = pl.program_id(0)
        li = lax.rem(i, 16) * MBLK + lax.broadcasted_iota(
            jnp.int32, (MBLK, 1), 0)
        mask = (li < N_NODES).astype(jnp.float32)
        consr = jax.nn.relu(a0[...] + brel[...] + rootc[...]) * mask
        varr = jax.nn.relu(b0[...] + brel[...] + rootv[...]) * mask
        part = jnp.concatenate([jnp.sum(consr, 0, keepdims=True),
                                jnp.sum(varr, 0, keepdims=True)], axis=0)
        sel = (i < 16).astype(jnp.float32)
        part4 = jnp.concatenate([part * sel, part * (1.0 - sel)], axis=0)

        @pl.when(i == 0)
        def _zero():
            out[...] = jnp.zeros_like(out)

        out[...] += part4

    def full(shape):
        return pl.BlockSpec(shape, lambda i: (0, 0))

    return pl.pallas_call(
        body,
        grid=(32,),
        in_specs=[pl.BlockSpec((MBLK, 4), lambda i: (i, 0)),
                  pl.BlockSpec((MBLK, 4), lambda i: (i, 0)),
                  pl.BlockSpec((MBLK, 4), lambda i: (i, 0)),
                  pl.BlockSpec((MBLK, 4), lambda i: (i, 0)),
                  full((1, 4))],
        out_specs=full((4, 4)),
        out_shape=jax.ShapeDtypeStruct((4, 4), jnp.float32),
    )


def _ln_row(x, g, b, eps=1e-5):
    m = jnp.mean(x, -1, keepdims=True)
    v = jnp.var(x, -1, keepdims=True)
    return (x - m) / jnp.sqrt(v + eps) * g + b


def kernel(constraint_features_s, edge_index_s, edge_attr_s,
           variable_features_s, bounds_s,
           constraint_features_t, edge_index_t, edge_attr_t,
           variable_features_t, bounds_t, params):
    p = params
    relu = jax.nn.relu
    e_pad = _edge_pad(edge_index_s.shape[1])

    c = p['ln_edge_b'][0]
    cons0row = relu(p['ln_cons_b'][0] * p['W_cons'][:, 0] + p['b_cons'])
    rootc1 = (cons0row @ p['Wroot1'].T).reshape(1, 8)
    tabC1row = (c * (cons0row @ p['Wrel1'].T)).reshape(1, 8)
    g6 = p['ln_var_g'].reshape(1, 6)
    b6 = p['ln_var_b'].reshape(1, 6)
    WvT = p['W_var'].T
    bv = p['b_var'].reshape(1, 32)
    weights = {
        1: (p['brel1'].reshape(1, 8), p['Wroot1'].T, c * p['Wrel1'].T),
        2: (p['brel2'].reshape(1, 4), p['Wroot2'].T, c * p['Wrel2'].T),
        3: (p['brel3'].reshape(1, 4), p['Wroot3'].T, c * p['Wrel3'].T),
    }
    zeros8 = jnp.zeros((N_PAD, 8), jnp.float32)
    zeros4 = jnp.zeros((N_PAD, 4), jnp.float32)
    tabC1 = jnp.tile(tabC1row, (2 * N_PAD, 1))

    stage0 = _tc_stage0()
    mid1 = _tc_stage_mid(8, 4, True)
    mid2 = _tc_stage_mid(4, 4, False)
    stage3 = _tc_stage3()
    pass8 = _sc_pass(8, e_pad)
    pass4 = _sc_pass(4, e_pad)

    def pad_idx(edge_index):
        e = edge_index.shape[1]
        padv = jnp.full((2, e_pad - e), DUMMY, jnp.int32)
        return jnp.concatenate([edge_index.astype(jnp.int32), padv], axis=1)

    ei_s = pad_idx(edge_index_s)
    ei_t = pad_idx(edge_index_t)
    idx0 = jnp.concatenate([ei_s[0], ei_t[0]]).reshape(-1, EB)
    idx1 = jnp.concatenate([ei_s[1], ei_t[1]]).reshape(-1, EB)

    def pad_vf(vf):
        return jnp.pad(vf, ((0, N_PAD - vf.shape[0]), (0, 0)))

    vfp = jnp.concatenate([pad_vf(variable_features_s),
                           pad_vf(variable_features_t)])

    rootv1, tabV1 = stage0(vfp, g6, b6, WvT, bv, weights[1][1], weights[1][2])
    aggA, aggB = pass8(idx0, idx1, tabV1, tabC1, zeros8)
    rootc2, rootv2, tabC2, tabV2 = mid1(
        aggA, aggB, rootc1, rootv1, weights[1][0],
        weights[2][1], weights[2][2])
    aggA, aggB = pass4(idx0, idx1, tabV2, tabC2, zeros4)
    rootc3, rootv3, tabC3, tabV3 = mid2(
        aggA, aggB, rootc2, rootv2, weights[2][0],
        weights[3][1], weights[3][2])
    aggA, aggB = pass4(idx0, idx1, tabV3, tabC3, zeros4)
    sums = stage3(aggA, aggB, rootc3, rootv3, weights[3][0])

    def bnd_row(bounds):
        return relu(_ln_row(bounds, p['ln_bnd_g'], p['ln_bnd_b'])
                    @ p['W_bnd'].T + p['b_bnd'])

    out0 = jnp.concatenate([sums[1:2] / N_NODES, sums[0:1] / N_NODES,
                            bnd_row(bounds_s)], axis=1)
    out1 = jnp.concatenate([sums[3:4] / N_NODES, sums[2:3] / N_NODES,
                            bnd_row(bounds_t)], axis=1)
    score0 = jnp.linalg.norm(out0, axis=1)
    score1 = jnp.linalg.norm(out1, axis=1)
    return jax.nn.sigmoid(-score0 + score1)


# speedup vs baseline: 38.6231x; 1.2973x over previous
import functools

import jax
import jax.numpy as jnp
from jax import lax
from jax.experimental import pallas as pl
from jax.experimental.pallas import tpu as pltpu
from jax.experimental.pallas import tpu_sc as plsc

N_NODES = 50000
N_PAD = 50048
NC, NS = 2, 16
EB = 128
RPC = 4
OUT_ROWS = N_PAD // NS
PR = 2 * N_PAD // 16
GBLK = PR // 2


@functools.lru_cache(maxsize=None)
def _sc_pass(d, n_edges):
    assert n_edges % EB == 0
    rows_pg = n_edges // EB
    rw_main = ((rows_pg + NS - 1) // NS + RPC - 1) // RPC * RPC
    rows_last = rows_pg - (NS - 1) * rw_main
    assert 2 * RPC <= rows_last <= rw_main and rows_last % RPC == 0
    nch_main = rw_main // RPC
    nch_last = rows_last // RPC
    mesh = plsc.VectorSubcoreMesh(core_axis_name="c", subcore_axis_name="s")

    def body(idx0_hbm, idx1_hbm, tabA_hbm, tabB_hbm, zeros_hbm,
             aggA_hbm, aggB_hbm,
             idx0_v, idx1_v, rowsA_v, rowsB_v,
             tabA_sh, tabB_sh, accA_sh, accB_sh,
             sem_i, sem_g, sem_s):
        c = lax.axis_index("c")
        s = lax.axis_index("s")
        src0 = c * N_PAD + s * OUT_ROWS
        dst0 = s * OUT_ROWS
        pltpu.sync_copy(tabA_hbm.at[pl.ds(src0, OUT_ROWS)],
                        tabA_sh.at[pl.ds(dst0, OUT_ROWS)])
        pltpu.sync_copy(tabB_hbm.at[pl.ds(src0, OUT_ROWS)],
                        tabB_sh.at[pl.ds(dst0, OUT_ROWS)])
        pltpu.sync_copy(zeros_hbm.at[pl.ds(dst0, OUT_ROWS)],
                        accA_sh.at[pl.ds(dst0, OUT_ROWS)])
        pltpu.sync_copy(zeros_hbm.at[pl.ds(dst0, OUT_ROWS)],
                        accB_sh.at[pl.ds(dst0, OUT_ROWS)])
        plsc.subcore_barrier()

        row_base = s * rw_main
        n_chunks = lax.select(s == NS - 1, nch_last, nch_main)

        def issue_idx(t, b):
            pltpu.async_copy(
                idx0_hbm.at[c, pl.ds(row_base + t * RPC, RPC)],
                idx0_v.at[b], sem_i)
            pltpu.async_copy(
                idx1_hbm.at[c, pl.ds(row_base + t * RPC, RPC)],
                idx1_v.at[b], sem_i)

        def wait_idx(b):
            pltpu.make_async_copy(idx0_hbm.at[0, pl.ds(0, RPC)],
                                  idx0_v.at[b], sem_i).wait()
            pltpu.make_async_copy(idx1_hbm.at[0, pl.ds(0, RPC)],
                                  idx1_v.at[b], sem_i).wait()

        def issue_gather(b):
            for j in range(RPC):
                pltpu.async_copy(tabA_sh.at[idx0_v.at[b, j]],
                                 rowsA_v.at[b, pl.ds(j * EB, EB)], sem_g)
                pltpu.async_copy(tabB_sh.at[idx1_v.at[b, j]],
                                 rowsB_v.at[b, pl.ds(j * EB, EB)], sem_g)

        def wait_gather(b):
            for j in range(RPC):
                pltpu.make_async_copy(tabA_sh.at[idx0_v.at[b, j]],
                                      rowsA_v.at[b, pl.ds(j * EB, EB)],
                                      sem_g).wait()
                pltpu.make_async_copy(tabB_sh.at[idx1_v.at[b, j]],
                                      rowsB_v.at[b, pl.ds(j * EB, EB)],
                                      sem_g).wait()

        def issue_scatter(b):
            for j in range(RPC):
                pltpu.async_copy(rowsA_v.at[b, pl.ds(j * EB, EB)],
                                 accA_sh.at[idx1_v.at[b, j]], sem_s, add=True)
                pltpu.async_copy(rowsB_v.at[b, pl.ds(j * EB, EB)],
                                 accB_sh.at[idx0_v.at[b, j]], sem_s, add=True)

        def wait_scatter(b):
            for j in range(RPC):
                pltpu.make_async_copy(rowsA_v.at[b, pl.ds(j * EB, EB)],
                                      accA_sh.at[idx1_v.at[b, j]],
                                      sem_s).wait()
                pltpu.make_async_copy(rowsB_v.at[b, pl.ds(j * EB, EB)],
                                      accB_sh.at[idx0_v.at[b, j]],
                                      sem_s).wait()

        issue_idx(0, 0)
        issue_idx(1, 1)
        wait_idx(0)
        issue_gather(0)

        def chunk_body(t, carry):
            cur = lax.rem(t, 2)

            def on(b):
                wait_gather(b)
                issue_scatter(b)

                @pl.when(t + 1 < n_chunks)
                def _g():
                    wait_idx(1 - b)
                    issue_gather(1 - b)

                wait_scatter(b)

                @pl.when(t + 2 < n_chunks)
                def _i():
                    issue_idx(t + 2, b)

            @pl.when(cur == 0)
            def _b0():
                on(0)

            @pl.when(cur == 1)
            def _b1():
                on(1)

            return carry

        lax.fori_loop(0, n_chunks, chunk_body, 0)
        plsc.subcore_barrier()
        out0 = c * N_PAD + s * OUT_ROWS
        pltpu.sync_copy(accA_sh.at[pl.ds(dst0, OUT_ROWS)],
                        aggA_hbm.at[pl.ds(out0, OUT_ROWS)])
        pltpu.sync_copy(accB_sh.at[pl.ds(dst0, OUT_ROWS)],
                        aggB_hbm.at[pl.ds(out0, OUT_ROWS)])

    return pl.kernel(
        body,
        out_type=(jax.ShapeDtypeStruct((2 * N_PAD, d), jnp.float32),
                  jax.ShapeDtypeStruct((2 * N_PAD, d), jnp.float32)),
        mesh=mesh,
        scratch_types=[
            pltpu.VMEM((2, RPC, EB), jnp.int32),
            pltpu.VMEM((2, RPC, EB), jnp.int32),
            pltpu.VMEM((2, RPC * EB, d), jnp.float32),
            pltpu.VMEM((2, RPC * EB, d), jnp.float32),
            pltpu.VMEM_SHARED((N_PAD, d), jnp.float32),
            pltpu.VMEM_SHARED((N_PAD, d), jnp.float32),
            pltpu.VMEM_SHARED((N_PAD, d), jnp.float32),
            pltpu.VMEM_SHARED((N_PAD, d), jnp.float32),
            pltpu.SemaphoreType.DMA,
            pltpu.SemaphoreType.DMA,
            pltpu.SemaphoreType.DMA,
        ],
        compiler_params=pltpu.CompilerParams(use_tc_tiling_on_sc=False),
    )


_DOT = functools.partial(jnp.dot, precision=lax.Precision.HIGHEST)


@functools.lru_cache(maxsize=None)
def _tc_stage0():
    def body(vf, G6, lmask, gP, bP, WvK, bvK, WrootK, cWrelK,
             rootv_o, tabv_o):
        x = vf[...]
        m = _DOT(x, G6[...])
        dd = (x - m) * lmask[...]
        v = _DOT(dd * dd, G6[...])
        xn = dd * lax.rsqrt(v + 1e-5) * gP[...] + bP[...]
        var0 = jax.nn.relu(_DOT(xn, WvK[...]) + bvK[...])
        rootv_o[...] = _DOT(var0, WrootK[...])
        tabv_o[...] = _DOT(var0, cWrelK[...])

    def full(shape):
        return pl.BlockSpec(shape, lambda i: (0, 0))

    return pl.pallas_call(
        body,
        grid=(2,),
        in_specs=[pl.BlockSpec((GBLK, 128), lambda i: (i, 0)),
                  full((128, 128)), full((1, 128)), full((1, 128)),
                  full((1, 128)), full((128, 512)), full((1, 512)),
                  full((512, 128)), full((512, 128))],
        out_specs=[pl.BlockSpec((GBLK, 128), lambda i: (i, 0))] * 2,
        out_shape=[jax.ShapeDtypeStruct((PR, 128), jnp.float32)] * 2,
    )


@functools.lru_cache(maxsize=None)
def _tc_stage_mid(w_in, rootc_bcast):
    def body(a0, b0, rootc, rootv, brel, WrootK, cWrelK,
             rootc_o, rootv_o, tabc_o, tabv_o):
        consr = jax.nn.relu(a0[...] + brel[...] + rootc[...])
        varr = jax.nn.relu(b0[...] + brel[...] + rootv[...])
        rootc_o[...] = _DOT(consr, WrootK[...])
        rootv_o[...] = _DOT(varr, WrootK[...])
        tabc_o[...] = _DOT(consr, cWrelK[...])
        tabv_o[...] = _DOT(varr, cWrelK[...])

    def full(shape):
        return pl.BlockSpec(shape, lambda i: (0, 0))

    rootc_spec = (full((1, w_in)) if rootc_bcast
                  else pl.BlockSpec((GBLK, w_in), lambda i: (i, 0)))
    return pl.pallas_call(
        body,
        grid=(2,),
        in_specs=[pl.BlockSpec((GBLK, w_in), lambda i: (i, 0)),
                  pl.BlockSpec((GBLK, w_in), lambda i: (i, 0)),
                  rootc_spec,
                  pl.BlockSpec((GBLK, w_in), lambda i: (i, 0)),
                  full((1, w_in)), full((w_in, 64)), full((w_in, 64))],
        out_specs=[pl.BlockSpec((GBLK, 64), lambda i: (i, 0))] * 4,
        out_shape=[jax.ShapeDtypeStruct((PR, 64), jnp.float32)] * 4,
    )


@functools.lru_cache(maxsize=None)
def _tc_stage3():
    def body(a0, b0, rootc, rootv, brel, out):
        i = pl.program_id(0)
        ri = lax.broadcasted_iota(jnp.int32, (GBLK, 1), 0)
        mask = (ri < (N_NODES // 16)).astype(jnp.float32)
        consr = jax.nn.relu(a0[...] + brel[...] + rootc[...]) * mask
        varr = jax.nn.relu(b0[...] + brel[...] + rootv[...]) * mask
        part = jnp.concatenate([jnp.sum(consr, 0, keepdims=True),
                                jnp.sum(varr, 0, keepdims=True)], axis=0)
        sel = (i == 0).astype(jnp.float32)
        part4 = jnp.concatenate([part * sel, part * (1.0 - sel)], axis=0)

        @pl.when(i == 0)
        def _zero():
            out[...] = jnp.zeros_like(out)

        out[...] += part4

    def full(shape):
        return pl.BlockSpec(shape, lambda i: (0, 0))

    return pl.pallas_call(
        body,
        grid=(2,),
        in_specs=[pl.BlockSpec((GBLK, 64), lambda i: (i, 0)),
                  pl.BlockSpec((GBLK, 64), lambda i: (i, 0)),
                  pl.BlockSpec((GBLK, 64), lambda i: (i, 0)),
                  pl.BlockSpec((GBLK, 64), lambda i: (i, 0)),
                  full((1, 64))],
        out_specs=full((4, 64)),
        out_shape=jax.ShapeDtypeStruct((4, 64), jnp.float32),
    )


def _ln_row(x, g, b, eps=1e-5):
    m = jnp.mean(x, -1, keepdims=True)
    v = jnp.var(x, -1, keepdims=True)
    return (x - m) / jnp.sqrt(v + eps) * g + b


def kernel(constraint_features_s, edge_index_s, edge_attr_s,
           variable_features_s, bounds_s,
           constraint_features_t, edge_index_t, edge_attr_t,
           variable_features_t, bounds_t, params):
    p = params
    relu = jax.nn.relu
    n_edges = edge_index_s.shape[1]
    rows_pg = n_edges // EB

    c = p['ln_edge_b'][0]
    cons0row = relu(p['ln_cons_b'][0] * p['W_cons'][:, 0] + p['b_cons'])
    rootc1 = cons0row @ p['Wroot1'].T
    tabC1row = c * (cons0row @ p['Wrel1'].T)
    eye16 = jnp.eye(16, dtype=jnp.float32)

    def kron16(w):
        return jnp.kron(eye16, w.astype(jnp.float32))

    def tile16(row):
        return jnp.tile(row.reshape(1, -1), (1, 16)).reshape(1, -1)

    G6 = kron16(jnp.ones((8, 8), jnp.float32) / 6.0)
    lmask = tile16(jnp.array([1, 1, 1, 1, 1, 1, 0, 0], jnp.float32))
    pad2 = lambda r: jnp.concatenate([r, jnp.zeros((2,), jnp.float32)])
    gP = tile16(pad2(p['ln_var_g']))
    bP = tile16(pad2(p['ln_var_b']))
    WvT8 = jnp.concatenate([p['W_var'].T,
                            jnp.zeros((2, 32), jnp.float32)])
    WvK = kron16(WvT8)
    bvK = tile16(p['b_var'])
    Wroot1K = kron16(p['Wroot1'].T)
    cWrel1K = kron16(c * p['Wrel1'].T)
    brel1P = tile16(p['brel1'])
    rootc1P = tile16(rootc1)
    Wroot2K = kron16(p['Wroot2'].T)
    cWrel2K = kron16(c * p['Wrel2'].T)
    brel2P = tile16(p['brel2'])
    Wroot3K = kron16(p['Wroot3'].T)
    cWrel3K = kron16(c * p['Wrel3'].T)
    brel3P = tile16(p['brel3'])
    zeros8 = jnp.zeros((N_PAD, 8), jnp.float32)
    zeros4 = jnp.zeros((N_PAD, 4), jnp.float32)
    tabC1 = jnp.tile(tile16(tabC1row), (PR, 1)).reshape(2 * N_PAD, 8)

    stage0 = _tc_stage0()
    mid1 = _tc_stage_mid(128, True)
    mid2 = _tc_stage_mid(64, False)
    stage3 = _tc_stage3()
    pass8 = _sc_pass(8, n_edges)
    pass4 = _sc_pass(4, n_edges)

    idx0 = jnp.stack([edge_index_s[0].astype(jnp.int32).reshape(rows_pg, EB),
                      edge_index_t[0].astype(jnp.int32).reshape(rows_pg, EB)])
    idx1 = jnp.stack([edge_index_s[1].astype(jnp.int32).reshape(rows_pg, EB),
                      edge_index_t[1].astype(jnp.int32).reshape(rows_pg, EB)])

    def pack_vf(vf):
        return jnp.pad(vf, ((0, N_PAD - vf.shape[0]), (0, 2)))

    vfp = jnp.concatenate([pack_vf(variable_features_s),
                           pack_vf(variable_features_t)]).reshape(PR, 128)

    rootv1, tabV1 = stage0(vfp, G6, lmask, gP, bP, WvK, bvK,
                           Wroot1K, cWrel1K)
    aggA, aggB = pass8(idx0, idx1, tabV1.reshape(2 * N_PAD, 8), tabC1,
                       zeros8)
    rootc2, rootv2, tabC2, tabV2 = mid1(
        aggA.reshape(PR, 128), aggB.reshape(PR, 128), rootc1P, rootv1,
        brel1P, Wroot2K, cWrel2K)
    aggA, aggB = pass4(idx0, idx1, tabV2.reshape(2 * N_PAD, 4),
                       tabC2.reshape(2 * N_PAD, 4), zeros4)
    rootc3, rootv3, tabC3, tabV3 = mid2(
        aggA.reshape(PR, 64), aggB.reshape(PR, 64), rootc2, rootv2,
        brel2P, Wroot3K, cWrel3K)
    aggA, aggB = pass4(idx0, idx1, tabV3.reshape(2 * N_PAD, 4),
                       tabC3.reshape(2 * N_PAD, 4), zeros4)
    sums = stage3(aggA.reshape(PR, 64), aggB.reshape(PR, 64),
                  rootc3, rootv3, brel3P)
    sums = jnp.sum(sums.reshape(4, 16, 4), axis=1) / N_NODES

    def bnd_row(bounds):
        return relu(_ln_row(bounds, p['ln_bnd_g'], p['ln_bnd_b'])
                    @ p['W_bnd'].T + p['b_bnd'])

    out0 = jnp.concatenate([sums[1:2], sums[0:1], bnd_row(bounds_s)], axis=1)
    out1 = jnp.concatenate([sums[3:4], sums[2:3], bnd_row(bounds_t)], axis=1)
    score0 = jnp.linalg.norm(out0, axis=1)
    score1 = jnp.linalg.norm(out1, axis=1)
    return jax.nn.sigmoid(-score0 + score1)


# speedup vs baseline: 41.3251x; 1.0700x over previous
import functools

import jax
import jax.numpy as jnp
from jax import lax
from jax.experimental import pallas as pl
from jax.experimental.pallas import tpu as pltpu
from jax.experimental.pallas import tpu_sc as plsc

N_NODES = 50000
N_PAD = 50048
NC, NS = 2, 16
EB = 128
RPC = 4
OUT_ROWS = N_PAD // NS
PR = 2 * N_PAD // 16
GBLK = PR // 2


@functools.lru_cache(maxsize=None)
def _sc_pass(d, n_edges):
    assert n_edges % EB == 0
    w = 16 * d
    rows_pg = n_edges // EB
    rw_main = ((rows_pg + NS - 1) // NS + RPC - 1) // RPC * RPC
    rows_last = rows_pg - (NS - 1) * rw_main
    assert 2 * RPC <= rows_last <= rw_main and rows_last % RPC == 0
    nch_main = rw_main // RPC
    nch_last = rows_last // RPC
    mesh = plsc.VectorSubcoreMesh(core_axis_name="c", subcore_axis_name="s")

    def body(ei_s_hbm, ei_t_hbm, tabA_hbm, tabB_hbm, zeros_hbm,
             aggA_hbm, aggB_hbm,
             idx0_v, idx1_v, rowsA_v, rowsB_v,
             tabA_sh, tabB_sh, accA_sh, accB_sh,
             sem_i, sem_g, sem_s):
        c = lax.axis_index("c")
        s = lax.axis_index("s")
        tabA_n = tabA_hbm
        tabB_n = tabB_hbm
        zeros_n = zeros_hbm
        aggA_n = aggA_hbm
        aggB_n = aggB_hbm
        src0 = c * N_PAD + s * OUT_ROWS
        dst0 = s * OUT_ROWS
        pltpu.sync_copy(tabA_n.at[pl.ds(src0, OUT_ROWS)],
                        tabA_sh.at[pl.ds(dst0, OUT_ROWS)])
        pltpu.sync_copy(tabB_n.at[pl.ds(src0, OUT_ROWS)],
                        tabB_sh.at[pl.ds(dst0, OUT_ROWS)])
        pltpu.sync_copy(zeros_n.at[pl.ds(dst0, OUT_ROWS)],
                        accA_sh.at[pl.ds(dst0, OUT_ROWS)])
        pltpu.sync_copy(zeros_n.at[pl.ds(dst0, OUT_ROWS)],
                        accB_sh.at[pl.ds(dst0, OUT_ROWS)])
        plsc.subcore_barrier()

        row_base = s * rw_main
        n_chunks = lax.select(s == NS - 1, nch_last, nch_main)

        def issue_idx(t, b):
            sl = pl.ds(row_base + t * RPC, RPC)

            @pl.when(c == 0)
            def _s():
                pltpu.async_copy(ei_s_hbm.at[0, sl], idx0_v.at[b], sem_i)
                pltpu.async_copy(ei_s_hbm.at[1, sl], idx1_v.at[b], sem_i)

            @pl.when(c == 1)
            def _t():
                pltpu.async_copy(ei_t_hbm.at[0, sl], idx0_v.at[b], sem_i)
                pltpu.async_copy(ei_t_hbm.at[1, sl], idx1_v.at[b], sem_i)

        def wait_idx(b):
            pltpu.make_async_copy(ei_s_hbm.at[0, pl.ds(0, RPC)],
                                  idx0_v.at[b], sem_i).wait()
            pltpu.make_async_copy(ei_s_hbm.at[0, pl.ds(0, RPC)],
                                  idx1_v.at[b], sem_i).wait()

        def issue_gather(b):
            for j in range(RPC):
                pltpu.async_copy(tabA_sh.at[idx0_v.at[b, j]],
                                 rowsA_v.at[b, pl.ds(j * EB, EB)], sem_g)
                pltpu.async_copy(tabB_sh.at[idx1_v.at[b, j]],
                                 rowsB_v.at[b, pl.ds(j * EB, EB)], sem_g)

        def wait_gather(b):
            for j in range(RPC):
                pltpu.make_async_copy(tabA_sh.at[idx0_v.at[b, j]],
                                      rowsA_v.at[b, pl.ds(j * EB, EB)],
                                      sem_g).wait()
                pltpu.make_async_copy(tabB_sh.at[idx1_v.at[b, j]],
                                      rowsB_v.at[b, pl.ds(j * EB, EB)],
                                      sem_g).wait()

        def issue_scatter(b):
            for j in range(RPC):
                pltpu.async_copy(rowsA_v.at[b, pl.ds(j * EB, EB)],
                                 accA_sh.at[idx1_v.at[b, j]], sem_s, add=True)
                pltpu.async_copy(rowsB_v.at[b, pl.ds(j * EB, EB)],
                                 accB_sh.at[idx0_v.at[b, j]], sem_s, add=True)

        def wait_scatter(b):
            for j in range(RPC):
                pltpu.make_async_copy(rowsA_v.at[b, pl.ds(j * EB, EB)],
                                      accA_sh.at[idx1_v.at[b, j]],
                                      sem_s).wait()
                pltpu.make_async_copy(rowsB_v.at[b, pl.ds(j * EB, EB)],
                                      accB_sh.at[idx0_v.at[b, j]],
                                      sem_s).wait()

        issue_idx(0, 0)
        issue_idx(1, 1)
        wait_idx(0)
        issue_gather(0)

        def chunk_body(t, carry):
            cur = lax.rem(t, 2)

            def on(b):
                wait_gather(b)
                issue_scatter(b)

                @pl.when(t + 1 < n_chunks)
                def _g():
                    wait_idx(1 - b)
                    issue_gather(1 - b)

                wait_scatter(b)

                @pl.when(t + 2 < n_chunks)
                def _i():
                    issue_idx(t + 2, b)

            @pl.when(cur == 0)
            def _b0():
                on(0)

            @pl.when(cur == 1)
            def _b1():
                on(1)

            return carry

        lax.fori_loop(0, n_chunks, chunk_body, 0)
        plsc.subcore_barrier()
        out0 = c * N_PAD + s * OUT_ROWS
        pltpu.sync_copy(accA_sh.at[pl.ds(dst0, OUT_ROWS)],
                        aggA_n.at[pl.ds(out0, OUT_ROWS)])
        pltpu.sync_copy(accB_sh.at[pl.ds(dst0, OUT_ROWS)],
                        aggB_n.at[pl.ds(out0, OUT_ROWS)])

    return pl.kernel(
        body,
        out_type=(jax.ShapeDtypeStruct((2 * N_PAD, d), jnp.float32),
                  jax.ShapeDtypeStruct((2 * N_PAD, d), jnp.float32)),
        mesh=mesh,
        scratch_types=[
            pltpu.VMEM((2, RPC, EB), jnp.int32),
            pltpu.VMEM((2, RPC, EB), jnp.int32),
            pltpu.VMEM((2, RPC * EB, d), jnp.float32),
            pltpu.VMEM((2, RPC * EB, d), jnp.float32),
            pltpu.VMEM_SHARED((N_PAD, d), jnp.float32),
            pltpu.VMEM_SHARED((N_PAD, d), jnp.float32),
            pltpu.VMEM_SHARED((N_PAD, d), jnp.float32),
            pltpu.VMEM_SHARED((N_PAD, d), jnp.float32),
            pltpu.SemaphoreType.DMA,
            pltpu.SemaphoreType.DMA,
            pltpu.SemaphoreType.DMA,
        ],
        compiler_params=pltpu.CompilerParams(use_tc_tiling_on_sc=False),
    )


_DOT = functools.partial(jnp.dot, precision=lax.Precision.HIGHEST)


@functools.lru_cache(maxsize=None)
def _tc_stage0():
    def body(vf, G6, lmask, gP, bP, WvK, bvK, WrootK, cWrelK,
             rootv_o, tabv_o):
        x = vf[...]
        m = _DOT(x, G6[...])
        dd = (x - m) * lmask[...]
        v = _DOT(dd * dd, G6[...])
        xn = dd * lax.rsqrt(v + 1e-5) * gP[...] + bP[...]
        var0 = jax.nn.relu(_DOT(xn, WvK[...]) + bvK[...])
        rootv_o[...] = _DOT(var0, WrootK[...])
        tabv_o[...] = _DOT(var0, cWrelK[...])

    def full(shape):
        return pl.BlockSpec(shape, lambda i: (0, 0))

    return pl.pallas_call(
        body,
        grid=(2,),
        in_specs=[pl.BlockSpec((GBLK, 128), lambda i: (i, 0)),
                  full((128, 128)), full((1, 128)), full((1, 128)),
                  full((1, 128)), full((128, 512)), full((1, 512)),
                  full((512, 128)), full((512, 128))],
        out_specs=[pl.BlockSpec((GBLK, 128), lambda i: (i, 0))] * 2,
        out_shape=[jax.ShapeDtypeStruct((PR, 128), jnp.float32)] * 2,
    )


@functools.lru_cache(maxsize=None)
def _tc_stage_mid(w_in, rootc_bcast):
    def body(a0, b0, rootc, rootv, brel, WrootK, cWrelK,
             rootc_o, rootv_o, tabc_o, tabv_o):
        consr = jax.nn.relu(a0[...] + brel[...] + rootc[...])
        varr = jax.nn.relu(b0[...] + brel[...] + rootv[...])
        rootc_o[...] = _DOT(consr, WrootK[...])
        rootv_o[...] = _DOT(varr, WrootK[...])
        tabc_o[...] = _DOT(consr, cWrelK[...])
        tabv_o[...] = _DOT(varr, cWrelK[...])

    def full(shape):
        return pl.BlockSpec(shape, lambda i: (0, 0))

    rootc_spec = (full((1, w_in)) if rootc_bcast
                  else pl.BlockSpec((GBLK, w_in), lambda i: (i, 0)))
    return pl.pallas_call(
        body,
        grid=(2,),
        in_specs=[pl.BlockSpec((GBLK, w_in), lambda i: (i, 0)),
                  pl.BlockSpec((GBLK, w_in), lambda i: (i, 0)),
                  rootc_spec,
                  pl.BlockSpec((GBLK, w_in), lambda i: (i, 0)),
                  full((1, w_in)), full((w_in, 64)), full((w_in, 64))],
        out_specs=[pl.BlockSpec((GBLK, 64), lambda i: (i, 0))] * 4,
        out_shape=[jax.ShapeDtypeStruct((PR, 64), jnp.float32)] * 4,
    )


@functools.lru_cache(maxsize=None)
def _tc_stage3():
    def body(a0, b0, rootc, rootv, brel, out):
        i = pl.program_id(0)
        ri = lax.broadcasted_iota(jnp.int32, (GBLK, 1), 0)
        mask = (ri < (N_NODES // 16)).astype(jnp.float32)
        consr = jax.nn.relu(a0[...] + brel[...] + rootc[...]) * mask
        varr = jax.nn.relu(b0[...] + brel[...] + rootv[...]) * mask
        part = jnp.concatenate([jnp.sum(consr, 0, keepdims=True),
                                jnp.sum(varr, 0, keepdims=True)], axis=0)
        sel = (i == 0).astype(jnp.float32)
        part4 = jnp.concatenate([part * sel, part * (1.0 - sel)], axis=0)

        @pl.when(i == 0)
        def _zero():
            out[...] = jnp.zeros_like(out)

        out[...] += part4

    def full(shape):
        return pl.BlockSpec(shape, lambda i: (0, 0))

    return pl.pallas_call(
        body,
        grid=(2,),
        in_specs=[pl.BlockSpec((GBLK, 64), lambda i: (i, 0)),
                  pl.BlockSpec((GBLK, 64), lambda i: (i, 0)),
                  pl.BlockSpec((GBLK, 64), lambda i: (i, 0)),
                  pl.BlockSpec((GBLK, 64), lambda i: (i, 0)),
                  full((1, 64))],
        out_specs=full((4, 64)),
        out_shape=jax.ShapeDtypeStruct((4, 64), jnp.float32),
    )


def _ln_row(x, g, b, eps=1e-5):
    m = jnp.mean(x, -1, keepdims=True)
    v = jnp.var(x, -1, keepdims=True)
    return (x - m) / jnp.sqrt(v + eps) * g + b


def kernel(constraint_features_s, edge_index_s, edge_attr_s,
           variable_features_s, bounds_s,
           constraint_features_t, edge_index_t, edge_attr_t,
           variable_features_t, bounds_t, params):
    p = params
    relu = jax.nn.relu
    n_edges = edge_index_s.shape[1]
    rows_pg = n_edges // EB

    c = p['ln_edge_b'][0]
    cons0row = relu(p['ln_cons_b'][0] * p['W_cons'][:, 0] + p['b_cons'])
    rootc1 = cons0row @ p['Wroot1'].T
    tabC1row = c * (cons0row @ p['Wrel1'].T)
    eye16 = jnp.eye(16, dtype=jnp.float32)

    def kron16(w):
        return jnp.kron(eye16, w.astype(jnp.float32))

    def tile16(row):
        return jnp.tile(row.reshape(1, -1), (1, 16)).reshape(1, -1)

    G6 = kron16(jnp.ones((8, 8), jnp.float32) / 6.0)
    lmask = tile16(jnp.array([1, 1, 1, 1, 1, 1, 0, 0], jnp.float32))
    pad2 = lambda r: jnp.concatenate([r, jnp.zeros((2,), jnp.float32)])
    gP = tile16(pad2(p['ln_var_g']))
    bP = tile16(pad2(p['ln_var_b']))
    WvT8 = jnp.concatenate([p['W_var'].T,
                            jnp.zeros((2, 32), jnp.float32)])
    WvK = kron16(WvT8)
    bvK = tile16(p['b_var'])
    Wroot1K = kron16(p['Wroot1'].T)
    cWrel1K = kron16(c * p['Wrel1'].T)
    brel1P = tile16(p['brel1'])
    rootc1P = tile16(rootc1)
    Wroot2K = kron16(p['Wroot2'].T)
    cWrel2K = kron16(c * p['Wrel2'].T)
    brel2P = tile16(p['brel2'])
    Wroot3K = kron16(p['Wroot3'].T)
    cWrel3K = kron16(c * p['Wrel3'].T)
    brel3P = tile16(p['brel3'])
    zeros8 = jnp.zeros((N_PAD, 8), jnp.float32)
    zeros4 = jnp.zeros((N_PAD, 4), jnp.float32)
    tabC1 = jnp.tile(tile16(tabC1row), (PR, 1)).reshape(2 * N_PAD, 8)

    stage0 = _tc_stage0()
    mid1 = _tc_stage_mid(128, True)
    mid2 = _tc_stage_mid(64, False)
    stage3 = _tc_stage3()
    pass8 = _sc_pass(8, n_edges)
    pass4 = _sc_pass(4, n_edges)

    ei_s = edge_index_s.astype(jnp.int32).reshape(2, rows_pg, EB)
    ei_t = edge_index_t.astype(jnp.int32).reshape(2, rows_pg, EB)

    def pack_vf(vf):
        return jnp.pad(vf, ((0, N_PAD - vf.shape[0]), (0, 2)))

    vfp = jnp.concatenate([pack_vf(variable_features_s),
                           pack_vf(variable_features_t)]).reshape(PR, 128)

    rootv1, tabV1 = stage0(vfp, G6, lmask, gP, bP, WvK, bvK,
                           Wroot1K, cWrel1K)
    aggA, aggB = pass8(ei_s, ei_t, tabV1.reshape(2 * N_PAD, 8), tabC1,
                       zeros8)
    rootc2, rootv2, tabC2, tabV2 = mid1(
        aggA.reshape(PR, 128), aggB.reshape(PR, 128), rootc1P, rootv1,
        brel1P, Wroot2K, cWrel2K)
    aggA, aggB = pass4(ei_s, ei_t, tabV2.reshape(2 * N_PAD, 4),
                       tabC2.reshape(2 * N_PAD, 4), zeros4)
    rootc3, rootv3, tabC3, tabV3 = mid2(
        aggA.reshape(PR, 64), aggB.reshape(PR, 64), rootc2, rootv2,
        brel2P, Wroot3K, cWrel3K)
    aggA, aggB = pass4(ei_s, ei_t, tabV3.reshape(2 * N_PAD, 4),
                       tabC3.reshape(2 * N_PAD, 4), zeros4)
    sums = stage3(aggA.reshape(PR, 64), aggB.reshape(PR, 64),
                  rootc3, rootv3, brel3P)
    sums = jnp.sum(sums.reshape(4, 16, 4), axis=1) / N_NODES

    def bnd_row(bounds):
        return relu(_ln_row(bounds, p['ln_bnd_g'], p['ln_bnd_b'])
                    @ p['W_bnd'].T + p['b_bnd'])

    out0 = jnp.concatenate([sums[1:2], sums[0:1], bnd_row(bounds_s)], axis=1)
    out1 = jnp.concatenate([sums[3:4], sums[2:3], bnd_row(bounds_t)], axis=1)
    score0 = jnp.linalg.norm(out0, axis=1)
    score1 = jnp.linalg.norm(out1, axis=1)
    return jax.nn.sigmoid(-score0 + score1)


# speedup vs baseline: 71.8920x; 1.7397x over previous
import functools

import jax
import jax.numpy as jnp
from jax import lax
from jax.experimental import pallas as pl
from jax.experimental.pallas import tpu as pltpu
from jax.experimental.pallas import tpu_sc as plsc

N_NODES = 50000
N_PAD = 50048
NC, NS = 2, 16
EB = 128
RPC = 4
OUT_ROWS = N_PAD // NS
PR = 2 * N_PAD // 16
GBLK = PR // 2


@functools.lru_cache(maxsize=None)
def _sc_pass(d, n_edges):
    assert n_edges % EB == 0
    w = 16 * d
    rows_pg = n_edges // EB
    rw_main = ((rows_pg + NS - 1) // NS + RPC - 1) // RPC * RPC
    rows_last = rows_pg - (NS - 1) * rw_main
    assert 2 * RPC <= rows_last <= rw_main and rows_last % RPC == 0
    nch_main = rw_main // RPC
    nch_last = rows_last // RPC
    mesh = plsc.VectorSubcoreMesh(core_axis_name="c", subcore_axis_name="s")

    def body(ei_s_hbm, ei_t_hbm, tabA_hbm, tabB_hbm, zeros_hbm,
             aggA_hbm, aggB_hbm,
             idx0_v, idx1_v, rowsA_v, rowsB_v,
             tabA_sh, tabB_sh, accA_sh, accB_sh,
             sem_i, sem_g, sem_s):
        c = lax.axis_index("c")
        s = lax.axis_index("s")
        tabA_n = tabA_hbm
        tabB_n = tabB_hbm
        zeros_n = zeros_hbm
        aggA_n = aggA_hbm
        aggB_n = aggB_hbm
        src0 = c * N_PAD + s * OUT_ROWS
        dst0 = s * OUT_ROWS
        pltpu.sync_copy(tabA_n.at[pl.ds(src0, OUT_ROWS)],
                        tabA_sh.at[pl.ds(dst0, OUT_ROWS)])
        pltpu.sync_copy(tabB_n.at[pl.ds(src0, OUT_ROWS)],
                        tabB_sh.at[pl.ds(dst0, OUT_ROWS)])
        pltpu.sync_copy(zeros_n.at[pl.ds(dst0, OUT_ROWS)],
                        accA_sh.at[pl.ds(dst0, OUT_ROWS)])
        pltpu.sync_copy(zeros_n.at[pl.ds(dst0, OUT_ROWS)],
                        accB_sh.at[pl.ds(dst0, OUT_ROWS)])
        plsc.subcore_barrier()

        row_base = s * rw_main
        n_chunks = lax.select(s == NS - 1, nch_last, nch_main)

        def issue_idx(t, b):
            sl = pl.ds(row_base + t * RPC, RPC)

            @pl.when(c == 0)
            def _s():
                pltpu.async_copy(ei_s_hbm.at[0, sl], idx0_v.at[b], sem_i)
                pltpu.async_copy(ei_s_hbm.at[1, sl], idx1_v.at[b], sem_i)

            @pl.when(c == 1)
            def _t():
                pltpu.async_copy(ei_t_hbm.at[0, sl], idx0_v.at[b], sem_i)
                pltpu.async_copy(ei_t_hbm.at[1, sl], idx1_v.at[b], sem_i)

        def wait_idx(b):
            pltpu.make_async_copy(ei_s_hbm.at[0, pl.ds(0, RPC)],
                                  idx0_v.at[b], sem_i).wait()
            pltpu.make_async_copy(ei_s_hbm.at[0, pl.ds(0, RPC)],
                                  idx1_v.at[b], sem_i).wait()

        def issue_gather(b):
            for j in range(RPC):
                pltpu.async_copy(tabA_sh.at[idx0_v.at[b, j]],
                                 rowsA_v.at[b, pl.ds(j * EB, EB)], sem_g)
                pltpu.async_copy(tabB_sh.at[idx1_v.at[b, j]],
                                 rowsB_v.at[b, pl.ds(j * EB, EB)], sem_g)

        def wait_gather(b):
            for j in range(RPC):
                pltpu.make_async_copy(tabA_sh.at[idx0_v.at[b, j]],
                                      rowsA_v.at[b, pl.ds(j * EB, EB)],
                                      sem_g).wait()
                pltpu.make_async_copy(tabB_sh.at[idx1_v.at[b, j]],
                                      rowsB_v.at[b, pl.ds(j * EB, EB)],
                                      sem_g).wait()

        def issue_scatter(b):
            for j in range(RPC):
                pltpu.async_copy(rowsA_v.at[b, pl.ds(j * EB, EB)],
                                 accA_sh.at[idx1_v.at[b, j]], sem_s, add=True)
                pltpu.async_copy(rowsB_v.at[b, pl.ds(j * EB, EB)],
                                 accB_sh.at[idx0_v.at[b, j]], sem_s, add=True)

        def wait_scatter(b):
            for j in range(RPC):
                pltpu.make_async_copy(rowsA_v.at[b, pl.ds(j * EB, EB)],
                                      accA_sh.at[idx1_v.at[b, j]],
                                      sem_s).wait()
                pltpu.make_async_copy(rowsB_v.at[b, pl.ds(j * EB, EB)],
                                      accB_sh.at[idx0_v.at[b, j]],
                                      sem_s).wait()

        issue_idx(0, 0)
        issue_idx(1, 1)
        wait_idx(0)
        issue_gather(0)

        def chunk_body(t, carry):
            cur = lax.rem(t, 2)

            def on(b):
                wait_gather(b)
                issue_scatter(b)

                @pl.when(t + 1 < n_chunks)
                def _g():
                    wait_idx(1 - b)
                    issue_gather(1 - b)

                wait_scatter(b)

                @pl.when(t + 2 < n_chunks)
                def _i():
                    issue_idx(t + 2, b)

            @pl.when(cur == 0)
            def _b0():
                on(0)

            @pl.when(cur == 1)
            def _b1():
                on(1)

            return carry

        lax.fori_loop(0, n_chunks, chunk_body, 0)
        plsc.subcore_barrier()
        out0 = c * N_PAD + s * OUT_ROWS
        pltpu.sync_copy(accA_sh.at[pl.ds(dst0, OUT_ROWS)],
                        aggA_n.at[pl.ds(out0, OUT_ROWS)])
        pltpu.sync_copy(accB_sh.at[pl.ds(dst0, OUT_ROWS)],
                        aggB_n.at[pl.ds(out0, OUT_ROWS)])

    return pl.kernel(
        body,
        out_type=(jax.ShapeDtypeStruct((2 * N_PAD, d), jnp.float32),
                  jax.ShapeDtypeStruct((2 * N_PAD, d), jnp.float32)),
        mesh=mesh,
        scratch_types=[
            pltpu.VMEM((2, RPC, EB), jnp.int32),
            pltpu.VMEM((2, RPC, EB), jnp.int32),
            pltpu.VMEM((2, RPC * EB, d), jnp.float32),
            pltpu.VMEM((2, RPC * EB, d), jnp.float32),
            pltpu.VMEM_SHARED((N_PAD, d), jnp.float32),
            pltpu.VMEM_SHARED((N_PAD, d), jnp.float32),
            pltpu.VMEM_SHARED((N_PAD, d), jnp.float32),
            pltpu.VMEM_SHARED((N_PAD, d), jnp.float32),
            pltpu.SemaphoreType.DMA,
            pltpu.SemaphoreType.DMA,
            pltpu.SemaphoreType.DMA,
        ],
        compiler_params=pltpu.CompilerParams(use_tc_tiling_on_sc=False),
    )


_DOT = functools.partial(jnp.dot, precision=lax.Precision.HIGHEST)


@functools.lru_cache(maxsize=None)
def _tc_stage0():
    def body(vf, G6, lmask, gP, bP, WvK, bvK, WrootK, cWrelK,
             rootv_o, tabv_o):
        x = vf[...]
        m = _DOT(x, G6[...])
        dd = (x - m) * lmask[...]
        v = _DOT(dd * dd, G6[...])
        xn = dd * lax.rsqrt(v + 1e-5) * gP[...] + bP[...]
        var0 = jax.nn.relu(_DOT(xn, WvK[...]) + bvK[...])
        rootv_o[...] = _DOT(var0, WrootK[...])
        tabv_o[...] = _DOT(var0, cWrelK[...])

    def full(shape):
        return pl.BlockSpec(shape, lambda i: (0, 0))

    return pl.pallas_call(
        body,
        grid=(2,),
        in_specs=[pl.BlockSpec((GBLK, 128), lambda i: (i, 0)),
                  full((128, 128)), full((1, 128)), full((1, 128)),
                  full((1, 128)), full((128, 512)), full((1, 512)),
                  full((512, 128)), full((512, 128))],
        out_specs=[pl.BlockSpec((GBLK, 128), lambda i: (i, 0))] * 2,
        out_shape=[jax.ShapeDtypeStruct((PR, 128), jnp.float32)] * 2,
    )


@functools.lru_cache(maxsize=None)
def _tc_stage_mid(rootc_bcast):
    def body(a0, b0, rootc, rootv, brel, WrootK, cWrelK,
             rootc_o, rootv_o, tabc_o, tabv_o):
        consr = jax.nn.relu(a0[...] + brel[...] + rootc[...])
        varr = jax.nn.relu(b0[...] + brel[...] + rootv[...])
        rootc_o[...] = _DOT(consr, WrootK[...])
        rootv_o[...] = _DOT(varr, WrootK[...])
        tabc_o[...] = _DOT(consr, cWrelK[...])
        tabv_o[...] = _DOT(varr, cWrelK[...])

    def full(shape):
        return pl.BlockSpec(shape, lambda i: (0, 0))

    rootc_spec = (full((1, 128)) if rootc_bcast
                  else pl.BlockSpec((GBLK, 128), lambda i: (i, 0)))
    return pl.pallas_call(
        body,
        grid=(2,),
        in_specs=[pl.BlockSpec((GBLK, 128), lambda i: (i, 0)),
                  pl.BlockSpec((GBLK, 128), lambda i: (i, 0)),
                  rootc_spec,
                  pl.BlockSpec((GBLK, 128), lambda i: (i, 0)),
                  full((1, 128)), full((128, 128)), full((128, 128))],
        out_specs=[pl.BlockSpec((GBLK, 128), lambda i: (i, 0))] * 4,
        out_shape=[jax.ShapeDtypeStruct((PR, 128), jnp.float32)] * 4,
    )


@functools.lru_cache(maxsize=None)
def _tc_stage3():
    def body(a0, b0, rootc, rootv, brel, out):
        i = pl.program_id(0)
        ri = lax.broadcasted_iota(jnp.int32, (GBLK, 1), 0)
        mask = (ri < (N_NODES // 16)).astype(jnp.float32)
        consr = jax.nn.relu(a0[...] + brel[...] + rootc[...]) * mask
        varr = jax.nn.relu(b0[...] + brel[...] + rootv[...]) * mask
        part = jnp.concatenate([jnp.sum(consr, 0, keepdims=True),
                                jnp.sum(varr, 0, keepdims=True)], axis=0)
        sel = (i == 0).astype(jnp.float32)
        part4 = jnp.concatenate([part * sel, part * (1.0 - sel)], axis=0)

        @pl.when(i == 0)
        def _zero():
            out[...] = jnp.zeros_like(out)

        out[...] += part4

    def full(shape):
        return pl.BlockSpec(shape, lambda i: (0, 0))

    return pl.pallas_call(
        body,
        grid=(2,),
        in_specs=[pl.BlockSpec((GBLK, 128), lambda i: (i, 0)),
                  pl.BlockSpec((GBLK, 128), lambda i: (i, 0)),
                  pl.BlockSpec((GBLK, 128), lambda i: (i, 0)),
                  pl.BlockSpec((GBLK, 128), lambda i: (i, 0)),
                  full((1, 128))],
        out_specs=full((4, 128)),
        out_shape=jax.ShapeDtypeStruct((4, 128), jnp.float32),
    )


def _ln_row(x, g, b, eps=1e-5):
    m = jnp.mean(x, -1, keepdims=True)
    v = jnp.var(x, -1, keepdims=True)
    return (x - m) / jnp.sqrt(v + eps) * g + b


def kernel(constraint_features_s, edge_index_s, edge_attr_s,
           variable_features_s, bounds_s,
           constraint_features_t, edge_index_t, edge_attr_t,
           variable_features_t, bounds_t, params):
    p = params
    relu = jax.nn.relu
    n_edges = edge_index_s.shape[1]
    rows_pg = n_edges // EB

    c = p['ln_edge_b'][0]
    cons0row = relu(p['ln_cons_b'][0] * p['W_cons'][:, 0] + p['b_cons'])
    rootc1 = cons0row @ p['Wroot1'].T
    tabC1row = c * (cons0row @ p['Wrel1'].T)
    eye16 = jnp.eye(16, dtype=jnp.float32)

    def kron16(w):
        return jnp.kron(eye16, w.astype(jnp.float32))

    def tile16(row):
        return jnp.tile(row.reshape(1, -1), (1, 16)).reshape(1, -1)

    G6 = kron16(jnp.ones((8, 8), jnp.float32) / 6.0)
    lmask = tile16(jnp.array([1, 1, 1, 1, 1, 1, 0, 0], jnp.float32))
    pad2 = lambda r: jnp.concatenate([r, jnp.zeros((2,), jnp.float32)])
    gP = tile16(pad2(p['ln_var_g']))
    bP = tile16(pad2(p['ln_var_b']))
    WvT8 = jnp.concatenate([p['W_var'].T,
                            jnp.zeros((2, 32), jnp.float32)])
    WvK = kron16(WvT8)
    bvK = tile16(p['b_var'])
    Wroot1K = kron16(p['Wroot1'].T)
    cWrel1K = kron16(c * p['Wrel1'].T)
    brel1P = tile16(p['brel1'])
    rootc1P = tile16(rootc1)
    def pad8x8(w):
        return jnp.zeros((8, 8), jnp.float32).at[:w.shape[0], :w.shape[1]].set(w)

    def padrow8(r):
        return jnp.concatenate([r, jnp.zeros((8 - r.shape[0],), jnp.float32)])

    Wroot2K = kron16(pad8x8(p['Wroot2'].T))
    cWrel2K = kron16(pad8x8(c * p['Wrel2'].T))
    brel2P = tile16(padrow8(p['brel2']))
    Wroot3K = kron16(pad8x8(p['Wroot3'].T))
    cWrel3K = kron16(pad8x8(c * p['Wrel3'].T))
    brel3P = tile16(padrow8(p['brel3']))
    zeros8 = jnp.zeros((N_PAD, 8), jnp.float32)
    tabC1 = jnp.tile(tile16(tabC1row), (PR, 1)).reshape(2 * N_PAD, 8)

    stage0 = _tc_stage0()
    mid1 = _tc_stage_mid(True)
    mid2 = _tc_stage_mid(False)
    stage3 = _tc_stage3()
    pass8 = _sc_pass(8, n_edges)

    ei_s = edge_index_s.astype(jnp.int32).reshape(2, rows_pg, EB)
    ei_t = edge_index_t.astype(jnp.int32).reshape(2, rows_pg, EB)

    def pack_vf(vf):
        return jnp.pad(vf, ((0, N_PAD - vf.shape[0]), (0, 2)))

    vfp = jnp.concatenate([pack_vf(variable_features_s),
                           pack_vf(variable_features_t)]).reshape(PR, 128)

    rootv1, tabV1 = stage0(vfp, G6, lmask, gP, bP, WvK, bvK,
                           Wroot1K, cWrel1K)
    aggA, aggB = pass8(ei_s, ei_t, tabV1.reshape(2 * N_PAD, 8), tabC1,
                       zeros8)
    rootc2, rootv2, tabC2, tabV2 = mid1(
        aggA.reshape(PR, 128), aggB.reshape(PR, 128), rootc1P, rootv1,
        brel1P, Wroot2K, cWrel2K)
    aggA, aggB = pass8(ei_s, ei_t, tabV2.reshape(2 * N_PAD, 8),
                       tabC2.reshape(2 * N_PAD, 8), zeros8)
    rootc3, rootv3, tabC3, tabV3 = mid2(
        aggA.reshape(PR, 128), aggB.reshape(PR, 128), rootc2, rootv2,
        brel2P, Wroot3K, cWrel3K)
    aggA, aggB = pass8(ei_s, ei_t, tabV3.reshape(2 * N_PAD, 8),
                       tabC3.reshape(2 * N_PAD, 8), zeros8)
    sums = stage3(aggA.reshape(PR, 128), aggB.reshape(PR, 128),
                  rootc3, rootv3, brel3P)
    sums = jnp.sum(sums.reshape(4, 16, 8), axis=1)[:, :4] / N_NODES

    def bnd_row(bounds):
        return relu(_ln_row(bounds, p['ln_bnd_g'], p['ln_bnd_b'])
                    @ p['W_bnd'].T + p['b_bnd'])

    out0 = jnp.concatenate([sums[1:2], sums[0:1], bnd_row(bounds_s)], axis=1)
    out1 = jnp.concatenate([sums[3:4], sums[2:3], bnd_row(bounds_t)], axis=1)
    score0 = jnp.linalg.norm(out0, axis=1)
    score1 = jnp.linalg.norm(out1, axis=1)
    return jax.nn.sigmoid(-score0 + score1)


# speedup vs baseline: 75.8527x; 1.0551x over previous
import functools

import jax
import jax.numpy as jnp
from jax import lax
from jax.experimental import pallas as pl
from jax.experimental.pallas import tpu as pltpu
from jax.experimental.pallas import tpu_sc as plsc

N_NODES = 50000
N_PAD = 50048
NC, NS = 2, 16
EB = 128
RPC = 6
OUT_ROWS = N_PAD // NS
PR = 2 * N_PAD // 16
GBLK = PR // 2


@functools.lru_cache(maxsize=None)
def _sc_pass(d, n_edges):
    assert n_edges % EB == 0
    w = 16 * d
    rows_pg = n_edges // EB
    rw_main = ((rows_pg + NS - 1) // NS + RPC - 1) // RPC * RPC
    rows_last = rows_pg - (NS - 1) * rw_main
    nch_main = rw_main // RPC
    nch_last = rows_last // RPC
    tail = rows_last - nch_last * RPC
    assert 2 * RPC <= rows_last <= rw_main and 0 <= tail < RPC
    mesh = plsc.VectorSubcoreMesh(core_axis_name="c", subcore_axis_name="s")

    def body(ei_s_hbm, ei_t_hbm, tabA_hbm, tabB_hbm, zeros_hbm,
             aggA_hbm, aggB_hbm,
             idx0_v, idx1_v, rowsA_v, rowsB_v,
             tabA_sh, tabB_sh, accA_sh, accB_sh,
             sem_i, sem_g, sem_s):
        c = lax.axis_index("c")
        s = lax.axis_index("s")
        tabA_n = tabA_hbm
        tabB_n = tabB_hbm
        zeros_n = zeros_hbm
        aggA_n = aggA_hbm
        aggB_n = aggB_hbm
        src0 = c * N_PAD + s * OUT_ROWS
        dst0 = s * OUT_ROWS
        pltpu.sync_copy(tabA_n.at[pl.ds(src0, OUT_ROWS)],
                        tabA_sh.at[pl.ds(dst0, OUT_ROWS)])
        pltpu.sync_copy(tabB_n.at[pl.ds(src0, OUT_ROWS)],
                        tabB_sh.at[pl.ds(dst0, OUT_ROWS)])
        pltpu.sync_copy(zeros_n.at[pl.ds(dst0, OUT_ROWS)],
                        accA_sh.at[pl.ds(dst0, OUT_ROWS)])
        pltpu.sync_copy(zeros_n.at[pl.ds(dst0, OUT_ROWS)],
                        accB_sh.at[pl.ds(dst0, OUT_ROWS)])
        plsc.subcore_barrier()

        row_base = s * rw_main
        n_chunks = lax.select(s == NS - 1, nch_last, nch_main)

        def issue_idx(t, b):
            sl = pl.ds(row_base + t * RPC, RPC)

            @pl.when(c == 0)
            def _s():
                pltpu.async_copy(ei_s_hbm.at[0, sl], idx0_v.at[b], sem_i)
                pltpu.async_copy(ei_s_hbm.at[1, sl], idx1_v.at[b], sem_i)

            @pl.when(c == 1)
            def _t():
                pltpu.async_copy(ei_t_hbm.at[0, sl], idx0_v.at[b], sem_i)
                pltpu.async_copy(ei_t_hbm.at[1, sl], idx1_v.at[b], sem_i)

        def wait_idx(b):
            pltpu.make_async_copy(ei_s_hbm.at[0, pl.ds(0, RPC)],
                                  idx0_v.at[b], sem_i).wait()
            pltpu.make_async_copy(ei_s_hbm.at[0, pl.ds(0, RPC)],
                                  idx1_v.at[b], sem_i).wait()

        def issue_gather(b):
            for j in range(RPC):
                pltpu.async_copy(tabA_sh.at[idx0_v.at[b, j]],
                                 rowsA_v.at[b, pl.ds(j * EB, EB)], sem_g)
                pltpu.async_copy(tabB_sh.at[idx1_v.at[b, j]],
                                 rowsB_v.at[b, pl.ds(j * EB, EB)], sem_g)

        def wait_gather(b):
            for j in range(RPC):
                pltpu.make_async_copy(tabA_sh.at[idx0_v.at[b, j]],
                                      rowsA_v.at[b, pl.ds(j * EB, EB)],
                                      sem_g).wait()
                pltpu.make_async_copy(tabB_sh.at[idx1_v.at[b, j]],
                                      rowsB_v.at[b, pl.ds(j * EB, EB)],
                                      sem_g).wait()

        def issue_scatter(b):
            for j in range(RPC):
                pltpu.async_copy(rowsA_v.at[b, pl.ds(j * EB, EB)],
                                 accA_sh.at[idx1_v.at[b, j]], sem_s, add=True)
                pltpu.async_copy(rowsB_v.at[b, pl.ds(j * EB, EB)],
                                 accB_sh.at[idx0_v.at[b, j]], sem_s, add=True)

        def wait_scatter(b):
            for j in range(RPC):
                pltpu.make_async_copy(rowsA_v.at[b, pl.ds(j * EB, EB)],
                                      accA_sh.at[idx1_v.at[b, j]],
                                      sem_s).wait()
                pltpu.make_async_copy(rowsB_v.at[b, pl.ds(j * EB, EB)],
                                      accB_sh.at[idx0_v.at[b, j]],
                                      sem_s).wait()

        issue_idx(0, 0)
        issue_idx(1, 1)
        wait_idx(0)
        issue_gather(0)

        def chunk_body(t, carry):
            cur = lax.rem(t, 2)

            def on(b):
                wait_gather(b)
                issue_scatter(b)

                @pl.when(t + 1 < n_chunks)
                def _g():
                    wait_idx(1 - b)
                    issue_gather(1 - b)

                wait_scatter(b)

                @pl.when(t + 2 < n_chunks)
                def _i():
                    issue_idx(t + 2, b)

            @pl.when(cur == 0)
            def _b0():
                on(0)

            @pl.when(cur == 1)
            def _b1():
                on(1)

            return carry

        lax.fori_loop(0, n_chunks, chunk_body, 0)
        if tail:
            @pl.when(s == NS - 1)
            def _tail():
                sl = pl.ds(row_base + nch_last * RPC, tail)
                tv = pl.ds(0, tail)

                @pl.when(c == 0)
                def _ts():
                    pltpu.async_copy(ei_s_hbm.at[0, sl], idx0_v.at[0, tv],
                                     sem_i)
                    pltpu.async_copy(ei_s_hbm.at[1, sl], idx1_v.at[0, tv],
                                     sem_i)

                @pl.when(c == 1)
                def _tt():
                    pltpu.async_copy(ei_t_hbm.at[0, sl], idx0_v.at[0, tv],
                                     sem_i)
                    pltpu.async_copy(ei_t_hbm.at[1, sl], idx1_v.at[0, tv],
                                     sem_i)

                pltpu.make_async_copy(ei_s_hbm.at[0, sl], idx0_v.at[0, tv],
                                      sem_i).wait()
                pltpu.make_async_copy(ei_s_hbm.at[0, sl], idx1_v.at[0, tv],
                                      sem_i).wait()
                for j in range(tail):
                    pltpu.async_copy(tabA_sh.at[idx0_v.at[0, j]],
                                     rowsA_v.at[0, pl.ds(j * EB, EB)], sem_g)
                    pltpu.async_copy(tabB_sh.at[idx1_v.at[0, j]],
                                     rowsB_v.at[0, pl.ds(j * EB, EB)], sem_g)
                for j in range(tail):
                    pltpu.make_async_copy(tabA_sh.at[idx0_v.at[0, j]],
                                          rowsA_v.at[0, pl.ds(j * EB, EB)],
                                          sem_g).wait()
                    pltpu.make_async_copy(tabB_sh.at[idx1_v.at[0, j]],
                                          rowsB_v.at[0, pl.ds(j * EB, EB)],
                                          sem_g).wait()
                for j in range(tail):
                    pltpu.async_copy(rowsA_v.at[0, pl.ds(j * EB, EB)],
                                     accA_sh.at[idx1_v.at[0, j]], sem_s,
                                     add=True)
                    pltpu.async_copy(rowsB_v.at[0, pl.ds(j * EB, EB)],
                                     accB_sh.at[idx0_v.at[0, j]], sem_s,
                                     add=True)
                for j in range(tail):
                    pltpu.make_async_copy(rowsA_v.at[0, pl.ds(j * EB, EB)],
                                          accA_sh.at[idx1_v.at[0, j]],
                                          sem_s).wait()
                    pltpu.make_async_copy(rowsB_v.at[0, pl.ds(j * EB, EB)],
                                          accB_sh.at[idx0_v.at[0, j]],
                                          sem_s).wait()
        plsc.subcore_barrier()
        out0 = c * N_PAD + s * OUT_ROWS
        pltpu.sync_copy(accA_sh.at[pl.ds(dst0, OUT_ROWS)],
                        aggA_n.at[pl.ds(out0, OUT_ROWS)])
        pltpu.sync_copy(accB_sh.at[pl.ds(dst0, OUT_ROWS)],
                        aggB_n.at[pl.ds(out0, OUT_ROWS)])

    return pl.kernel(
        body,
        out_type=(jax.ShapeDtypeStruct((2 * N_PAD, d), jnp.float32),
                  jax.ShapeDtypeStruct((2 * N_PAD, d), jnp.float32)),
        mesh=mesh,
        scratch_types=[
            pltpu.VMEM((2, RPC, EB), jnp.int32),
            pltpu.VMEM((2, RPC, EB), jnp.int32),
            pltpu.VMEM((2, RPC * EB, d), jnp.float32),
            pltpu.VMEM((2, RPC * EB, d), jnp.float32),
            pltpu.VMEM_SHARED((N_PAD, d), jnp.float32),
            pltpu.VMEM_SHARED((N_PAD, d), jnp.float32),
            pltpu.VMEM_SHARED((N_PAD, d), jnp.float32),
            pltpu.VMEM_SHARED((N_PAD, d), jnp.float32),
            pltpu.SemaphoreType.DMA,
            pltpu.SemaphoreType.DMA,
            pltpu.SemaphoreType.DMA,
        ],
        compiler_params=pltpu.CompilerParams(use_tc_tiling_on_sc=False),
    )


_DOT = functools.partial(jnp.dot, precision=lax.Precision.HIGHEST)


@functools.lru_cache(maxsize=None)
def _tc_stage0():
    def body(vf, G6, lmask, gP, bP, WvK, bvK, WrootK, cWrelK,
             rootv_o, tabv_o):
        x = vf[...]
        m = _DOT(x, G6[...])
        dd = (x - m) * lmask[...]
        v = _DOT(dd * dd, G6[...])
        xn = dd * lax.rsqrt(v + 1e-5) * gP[...] + bP[...]
        var0 = jax.nn.relu(_DOT(xn, WvK[...]) + bvK[...])
        rootv_o[...] = _DOT(var0, WrootK[...])
        tabv_o[...] = _DOT(var0, cWrelK[...])

    def full(shape):
        return pl.BlockSpec(shape, lambda i: (0, 0))

    return pl.pallas_call(
        body,
        grid=(2,),
        in_specs=[pl.BlockSpec((GBLK, 128), lambda i: (i, 0)),
                  full((128, 128)), full((1, 128)), full((1, 128)),
                  full((1, 128)), full((128, 512)), full((1, 512)),
                  full((512, 128)), full((512, 128))],
        out_specs=[pl.BlockSpec((GBLK, 128), lambda i: (i, 0))] * 2,
        out_shape=[jax.ShapeDtypeStruct((PR, 128), jnp.float32)] * 2,
    )


@functools.lru_cache(maxsize=None)
def _tc_stage_mid(rootc_bcast):
    def body(a0, b0, rootc, rootv, brel, WrootK, cWrelK,
             rootc_o, rootv_o, tabc_o, tabv_o):
        consr = jax.nn.relu(a0[...] + brel[...] + rootc[...])
        varr = jax.nn.relu(b0[...] + brel[...] + rootv[...])
        rootc_o[...] = _DOT(consr, WrootK[...])
        rootv_o[...] = _DOT(varr, WrootK[...])
        tabc_o[...] = _DOT(consr, cWrelK[...])
        tabv_o[...] = _DOT(varr, cWrelK[...])

    def full(shape):
        return pl.BlockSpec(shape, lambda i: (0, 0))

    rootc_spec = (full((1, 128)) if rootc_bcast
                  else pl.BlockSpec((GBLK, 128), lambda i: (i, 0)))
    return pl.pallas_call(
        body,
        grid=(2,),
        in_specs=[pl.BlockSpec((GBLK, 128), lambda i: (i, 0)),
                  pl.BlockSpec((GBLK, 128), lambda i: (i, 0)),
                  rootc_spec,
                  pl.BlockSpec((GBLK, 128), lambda i: (i, 0)),
                  full((1, 128)), full((128, 128)), full((128, 128))],
        out_specs=[pl.BlockSpec((GBLK, 128), lambda i: (i, 0))] * 4,
        out_shape=[jax.ShapeDtypeStruct((PR, 128), jnp.float32)] * 4,
    )


@functools.lru_cache(maxsize=None)
def _tc_stage3():
    def body(a0, b0, rootc, rootv, brel, out):
        i = pl.program_id(0)
        ri = lax.broadcasted_iota(jnp.int32, (GBLK, 1), 0)
        mask = (ri < (N_NODES // 16)).astype(jnp.float32)
        consr = jax.nn.relu(a0[...] + brel[...] + rootc[...]) * mask
        varr = jax.nn.relu(b0[...] + brel[...] + rootv[...]) * mask
        part = jnp.concatenate([jnp.sum(consr, 0, keepdims=True),
                                jnp.sum(varr, 0, keepdims=True)], axis=0)
        sel = (i == 0).astype(jnp.float32)
        part4 = jnp.concatenate([part * sel, part * (1.0 - sel)], axis=0)

        @pl.when(i == 0)
        def _zero():
            out[...] = jnp.zeros_like(out)

        out[...] += part4

    def full(shape):
        return pl.BlockSpec(shape, lambda i: (0, 0))

    return pl.pallas_call(
        body,
        grid=(2,),
        in_specs=[pl.BlockSpec((GBLK, 128), lambda i: (i, 0)),
                  pl.BlockSpec((GBLK, 128), lambda i: (i, 0)),
                  pl.BlockSpec((GBLK, 128), lambda i: (i, 0)),
                  pl.BlockSpec((GBLK, 128), lambda i: (i, 0)),
                  full((1, 128))],
        out_specs=full((4, 128)),
        out_shape=jax.ShapeDtypeStruct((4, 128), jnp.float32),
    )


def _ln_row(x, g, b, eps=1e-5):
    m = jnp.mean(x, -1, keepdims=True)
    v = jnp.var(x, -1, keepdims=True)
    return (x - m) / jnp.sqrt(v + eps) * g + b


def kernel(constraint_features_s, edge_index_s, edge_attr_s,
           variable_features_s, bounds_s,
           constraint_features_t, edge_index_t, edge_attr_t,
           variable_features_t, bounds_t, params):
    p = params
    relu = jax.nn.relu
    n_edges = edge_index_s.shape[1]
    rows_pg = n_edges // EB

    c = p['ln_edge_b'][0]
    cons0row = relu(p['ln_cons_b'][0] * p['W_cons'][:, 0] + p['b_cons'])
    rootc1 = cons0row @ p['Wroot1'].T
    tabC1row = c * (cons0row @ p['Wrel1'].T)
    eye16 = jnp.eye(16, dtype=jnp.float32)

    def kron16(w):
        return jnp.kron(eye16, w.astype(jnp.float32))

    def tile16(row):
        return jnp.tile(row.reshape(1, -1), (1, 16)).reshape(1, -1)

    G6 = kron16(jnp.ones((8, 8), jnp.float32) / 6.0)
    lmask = tile16(jnp.array([1, 1, 1, 1, 1, 1, 0, 0], jnp.float32))
    pad2 = lambda r: jnp.concatenate([r, jnp.zeros((2,), jnp.float32)])
    gP = tile16(pad2(p['ln_var_g']))
    bP = tile16(pad2(p['ln_var_b']))
    WvT8 = jnp.concatenate([p['W_var'].T,
                            jnp.zeros((2, 32), jnp.float32)])
    WvK = kron16(WvT8)
    bvK = tile16(p['b_var'])
    Wroot1K = kron16(p['Wroot1'].T)
    cWrel1K = kron16(c * p['Wrel1'].T)
    brel1P = tile16(p['brel1'])
    rootc1P = tile16(rootc1)
    def pad8x8(w):
        return jnp.zeros((8, 8), jnp.float32).at[:w.shape[0], :w.shape[1]].set(w)

    def padrow8(r):
        return jnp.concatenate([r, jnp.zeros((8 - r.shape[0],), jnp.float32)])

    Wroot2K = kron16(pad8x8(p['Wroot2'].T))
    cWrel2K = kron16(pad8x8(c * p['Wrel2'].T))
    brel2P = tile16(padrow8(p['brel2']))
    Wroot3K = kron16(pad8x8(p['Wroot3'].T))
    cWrel3K = kron16(pad8x8(c * p['Wrel3'].T))
    brel3P = tile16(padrow8(p['brel3']))
    zeros8 = jnp.zeros((N_PAD, 8), jnp.float32)
    tabC1 = jnp.tile(tile16(tabC1row), (PR, 1)).reshape(2 * N_PAD, 8)

    stage0 = _tc_stage0()
    mid1 = _tc_stage_mid(True)
    mid2 = _tc_stage_mid(False)
    stage3 = _tc_stage3()
    pass8 = _sc_pass(8, n_edges)

    ei_s = edge_index_s.astype(jnp.int32).reshape(2, rows_pg, EB)
    ei_t = edge_index_t.astype(jnp.int32).reshape(2, rows_pg, EB)

    def pack_vf(vf):
        return jnp.pad(vf, ((0, N_PAD - vf.shape[0]), (0, 2)))

    vfp = jnp.concatenate([pack_vf(variable_features_s),
                           pack_vf(variable_features_t)]).reshape(PR, 128)

    rootv1, tabV1 = stage0(vfp, G6, lmask, gP, bP, WvK, bvK,
                           Wroot1K, cWrel1K)
    aggA, aggB = pass8(ei_s, ei_t, tabV1.reshape(2 * N_PAD, 8), tabC1,
                       zeros8)
    rootc2, rootv2, tabC2, tabV2 = mid1(
        aggA.reshape(PR, 128), aggB.reshape(PR, 128), rootc1P, rootv1,
        brel1P, Wroot2K, cWrel2K)
    aggA, aggB = pass8(ei_s, ei_t, tabV2.reshape(2 * N_PAD, 8),
                       tabC2.reshape(2 * N_PAD, 8), zeros8)
    rootc3, rootv3, tabC3, tabV3 = mid2(
        aggA.reshape(PR, 128), aggB.reshape(PR, 128), rootc2, rootv2,
        brel2P, Wroot3K, cWrel3K)
    aggA, aggB = pass8(ei_s, ei_t, tabV3.reshape(2 * N_PAD, 8),
                       tabC3.reshape(2 * N_PAD, 8), zeros8)
    sums = stage3(aggA.reshape(PR, 128), aggB.reshape(PR, 128),
                  rootc3, rootv3, brel3P)
    sums = jnp.sum(sums.reshape(4, 16, 8), axis=1)[:, :4] / N_NODES

    def bnd_row(bounds):
        return relu(_ln_row(bounds, p['ln_bnd_g'], p['ln_bnd_b'])
                    @ p['W_bnd'].T + p['b_bnd'])

    out0 = jnp.concatenate([sums[1:2], sums[0:1], bnd_row(bounds_s)], axis=1)
    out1 = jnp.concatenate([sums[3:4], sums[2:3], bnd_row(bounds_t)], axis=1)
    score0 = jnp.linalg.norm(out0, axis=1)
    score1 = jnp.linalg.norm(out1, axis=1)
    return jax.nn.sigmoid(-score0 + score1)


# speedup vs baseline: 77.8191x; 1.0259x over previous
import functools

import jax
import jax.numpy as jnp
from jax import lax
from jax.experimental import pallas as pl
from jax.experimental.pallas import tpu as pltpu
from jax.experimental.pallas import tpu_sc as plsc

N_NODES = 50000
N_PAD = 50048
NC, NS = 2, 16
EB = 128
RPC = 6
OUT_ROWS = N_PAD // NS
PR = 2 * N_PAD // 16
GBLK = PR // 2


@functools.lru_cache(maxsize=None)
def _sc_pass(d, n_edges, const_b=False):
    assert n_edges % EB == 0
    w = 16 * d
    rows_pg = n_edges // EB
    rw_main = ((rows_pg + NS - 1) // NS + RPC - 1) // RPC * RPC
    rows_last = rows_pg - (NS - 1) * rw_main
    nch_main = rw_main // RPC
    nch_last = rows_last // RPC
    tail = rows_last - nch_last * RPC
    assert 2 * RPC <= rows_last <= rw_main and 0 <= tail < RPC
    mesh = plsc.VectorSubcoreMesh(core_axis_name="c", subcore_axis_name="s")

    def body(ei_s_hbm, ei_t_hbm, tabA_hbm, tabB_hbm, zeros_hbm,
             aggA_hbm, aggB_hbm,
             idx0_v, idx1_v, rowsA_v, rowsB_v, constB_v,
             tabA_sh, tabB_sh, accA_sh, accB_sh,
             sem_i, sem_g, sem_s):
        c = lax.axis_index("c")
        s = lax.axis_index("s")
        tabA_n = tabA_hbm
        tabB_n = tabB_hbm
        zeros_n = zeros_hbm
        aggA_n = aggA_hbm
        aggB_n = aggB_hbm
        src0 = c * N_PAD + s * OUT_ROWS
        dst0 = s * OUT_ROWS
        pltpu.sync_copy(tabA_n.at[pl.ds(src0, OUT_ROWS)],
                        tabA_sh.at[pl.ds(dst0, OUT_ROWS)])
        if const_b:
            pltpu.sync_copy(tabB_hbm, constB_v)
        else:
            pltpu.sync_copy(tabB_n.at[pl.ds(src0, OUT_ROWS)],
                            tabB_sh.at[pl.ds(dst0, OUT_ROWS)])
        pltpu.sync_copy(zeros_n.at[pl.ds(dst0, OUT_ROWS)],
                        accA_sh.at[pl.ds(dst0, OUT_ROWS)])
        pltpu.sync_copy(zeros_n.at[pl.ds(dst0, OUT_ROWS)],
                        accB_sh.at[pl.ds(dst0, OUT_ROWS)])
        plsc.subcore_barrier()

        row_base = s * rw_main
        n_chunks = lax.select(s == NS - 1, nch_last, nch_main)

        def issue_idx(t, b):
            sl = pl.ds(row_base + t * RPC, RPC)

            @pl.when(c == 0)
            def _s():
                pltpu.async_copy(ei_s_hbm.at[0, sl], idx0_v.at[b], sem_i)
                pltpu.async_copy(ei_s_hbm.at[1, sl], idx1_v.at[b], sem_i)

            @pl.when(c == 1)
            def _t():
                pltpu.async_copy(ei_t_hbm.at[0, sl], idx0_v.at[b], sem_i)
                pltpu.async_copy(ei_t_hbm.at[1, sl], idx1_v.at[b], sem_i)

        def wait_idx(b):
            pltpu.make_async_copy(ei_s_hbm.at[0, pl.ds(0, RPC)],
                                  idx0_v.at[b], sem_i).wait()
            pltpu.make_async_copy(ei_s_hbm.at[0, pl.ds(0, RPC)],
                                  idx1_v.at[b], sem_i).wait()

        def issue_gather(b):
            for j in range(RPC):
                pltpu.async_copy(tabA_sh.at[idx0_v.at[b, j]],
                                 rowsA_v.at[b, pl.ds(j * EB, EB)], sem_g)
                if not const_b:
                    pltpu.async_copy(tabB_sh.at[idx1_v.at[b, j]],
                                     rowsB_v.at[b, pl.ds(j * EB, EB)], sem_g)

        def wait_gather(b):
            for j in range(RPC):
                pltpu.make_async_copy(tabA_sh.at[idx0_v.at[b, j]],
                                      rowsA_v.at[b, pl.ds(j * EB, EB)],
                                      sem_g).wait()
                if not const_b:
                    pltpu.make_async_copy(tabB_sh.at[idx1_v.at[b, j]],
                                          rowsB_v.at[b, pl.ds(j * EB, EB)],
                                          sem_g).wait()

        def _srcB(b, j):
            if const_b:
                return constB_v
            return rowsB_v.at[b, pl.ds(j * EB, EB)]

        def issue_scatter(b):
            for j in range(RPC):
                pltpu.async_copy(rowsA_v.at[b, pl.ds(j * EB, EB)],
                                 accA_sh.at[idx1_v.at[b, j]], sem_s, add=True)
                pltpu.async_copy(_srcB(b, j),
                                 accB_sh.at[idx0_v.at[b, j]], sem_s, add=True)

        def wait_scatter(b):
            for j in range(RPC):
                pltpu.make_async_copy(rowsA_v.at[b, pl.ds(j * EB, EB)],
                                      accA_sh.at[idx1_v.at[b, j]],
                                      sem_s).wait()
                pltpu.make_async_copy(_srcB(b, j),
                                      accB_sh.at[idx0_v.at[b, j]],
                                      sem_s).wait()

        issue_idx(0, 0)
        issue_idx(1, 1)
        wait_idx(0)
        issue_gather(0)

        def chunk_body(t, carry):
            cur = lax.rem(t, 2)

            def on(b):
                wait_gather(b)
                issue_scatter(b)

                @pl.when(t + 1 < n_chunks)
                def _g():
                    wait_idx(1 - b)
                    issue_gather(1 - b)

                wait_scatter(b)

                @pl.when(t + 2 < n_chunks)
                def _i():
                    issue_idx(t + 2, b)

            @pl.when(cur == 0)
            def _b0():
                on(0)

            @pl.when(cur == 1)
            def _b1():
                on(1)

            return carry

        lax.fori_loop(0, n_chunks, chunk_body, 0)
        if tail:
            @pl.when(s == NS - 1)
            def _tail():
                sl = pl.ds(row_base + nch_last * RPC, tail)
                tv = pl.ds(0, tail)

                @pl.when(c == 0)
                def _ts():
                    pltpu.async_copy(ei_s_hbm.at[0, sl], idx0_v.at[0, tv],
                                     sem_i)
                    pltpu.async_copy(ei_s_hbm.at[1, sl], idx1_v.at[0, tv],
                                     sem_i)

                @pl.when(c == 1)
                def _tt():
                    pltpu.async_copy(ei_t_hbm.at[0, sl], idx0_v.at[0, tv],
                                     sem_i)
                    pltpu.async_copy(ei_t_hbm.at[1, sl], idx1_v.at[0, tv],
                                     sem_i)

                pltpu.make_async_copy(ei_s_hbm.at[0, sl], idx0_v.at[0, tv],
                                      sem_i).wait()
                pltpu.make_async_copy(ei_s_hbm.at[0, sl], idx1_v.at[0, tv],
                                      sem_i).wait()
                for j in range(tail):
                    pltpu.async_copy(tabA_sh.at[idx0_v.at[0, j]],
                                     rowsA_v.at[0, pl.ds(j * EB, EB)], sem_g)
                    if not const_b:
                        pltpu.async_copy(tabB_sh.at[idx1_v.at[0, j]],
                                         rowsB_v.at[0, pl.ds(j * EB, EB)],
                                         sem_g)
                for j in range(tail):
                    pltpu.make_async_copy(tabA_sh.at[idx0_v.at[0, j]],
                                          rowsA_v.at[0, pl.ds(j * EB, EB)],
                                          sem_g).wait()
                    if not const_b:
                        pltpu.make_async_copy(
                            tabB_sh.at[idx1_v.at[0, j]],
                            rowsB_v.at[0, pl.ds(j * EB, EB)], sem_g).wait()
                for j in range(tail):
                    pltpu.async_copy(rowsA_v.at[0, pl.ds(j * EB, EB)],
                                     accA_sh.at[idx1_v.at[0, j]], sem_s,
                                     add=True)
                    pltpu.async_copy(_srcB(0, j),
                                     accB_sh.at[idx0_v.at[0, j]], sem_s,
                                     add=True)
                for j in range(tail):
                    pltpu.make_async_copy(rowsA_v.at[0, pl.ds(j * EB, EB)],
                                          accA_sh.at[idx1_v.at[0, j]],
                                          sem_s).wait()
                    pltpu.make_async_copy(_srcB(0, j),
                                          accB_sh.at[idx0_v.at[0, j]],
                                          sem_s).wait()
        plsc.subcore_barrier()
        out0 = c * N_PAD + s * OUT_ROWS
        pltpu.sync_copy(accA_sh.at[pl.ds(dst0, OUT_ROWS)],
                        aggA_n.at[pl.ds(out0, OUT_ROWS)])
        pltpu.sync_copy(accB_sh.at[pl.ds(dst0, OUT_ROWS)],
                        aggB_n.at[pl.ds(out0, OUT_ROWS)])

    return pl.kernel(
        body,
        out_type=(jax.ShapeDtypeStruct((2 * N_PAD, d), jnp.float32),
                  jax.ShapeDtypeStruct((2 * N_PAD, d), jnp.float32)),
        mesh=mesh,
        scratch_types=[
            pltpu.VMEM((2, RPC, EB), jnp.int32),
            pltpu.VMEM((2, RPC, EB), jnp.int32),
            pltpu.VMEM((2, RPC * EB, d), jnp.float32),
            pltpu.VMEM((2, RPC * EB, d) if not const_b else (1, d),
                       jnp.float32),
            pltpu.VMEM((EB, d), jnp.float32),
            pltpu.VMEM_SHARED((N_PAD, d), jnp.float32),
            pltpu.VMEM_SHARED((N_PAD, d) if not const_b else (1, d),
                              jnp.float32),
            pltpu.VMEM_SHARED((N_PAD, d), jnp.float32),
            pltpu.VMEM_SHARED((N_PAD, d), jnp.float32),
            pltpu.SemaphoreType.DMA,
            pltpu.SemaphoreType.DMA,
            pltpu.SemaphoreType.DMA,
        ],
        compiler_params=pltpu.CompilerParams(use_tc_tiling_on_sc=False),
    )


_DOT = functools.partial(jnp.dot, precision=lax.Precision.HIGHEST)


@functools.lru_cache(maxsize=None)
def _tc_stage0():
    def body(vf, G6, lmask, gP, bP, WvK, bvK, WrootK, cWrelK,
             rootv_o, tabv_o):
        x = vf[...]
        m = _DOT(x, G6[...])
        dd = (x - m) * lmask[...]
        v = _DOT(dd * dd, G6[...])
        xn = dd * lax.rsqrt(v + 1e-5) * gP[...] + bP[...]
        var0 = jax.nn.relu(_DOT(xn, WvK[...]) + bvK[...])
        rootv_o[...] = _DOT(var0, WrootK[...])
        tabv_o[...] = _DOT(var0, cWrelK[...])

    def full(shape):
        return pl.BlockSpec(shape, lambda i: (0, 0))

    return pl.pallas_call(
        body,
        grid=(2,),
        in_specs=[pl.BlockSpec((GBLK, 128), lambda i: (i, 0)),
                  full((128, 128)), full((1, 128)), full((1, 128)),
                  full((1, 128)), full((128, 512)), full((1, 512)),
                  full((512, 128)), full((512, 128))],
        out_specs=[pl.BlockSpec((GBLK, 128), lambda i: (i, 0))] * 2,
        out_shape=[jax.ShapeDtypeStruct((PR, 128), jnp.float32)] * 2,
    )


@functools.lru_cache(maxsize=None)
def _tc_stage_mid(rootc_bcast):
    def body(a0, b0, rootc, rootv, brel, WrootK, cWrelK,
             rootc_o, rootv_o, tabc_o, tabv_o):
        consr = jax.nn.relu(a0[...] + brel[...] + rootc[...])
        varr = jax.nn.relu(b0[...] + brel[...] + rootv[...])
        rootc_o[...] = _DOT(consr, WrootK[...])
        rootv_o[...] = _DOT(varr, WrootK[...])
        tabc_o[...] = _DOT(consr, cWrelK[...])
        tabv_o[...] = _DOT(varr, cWrelK[...])

    def full(shape):
        return pl.BlockSpec(shape, lambda i: (0, 0))

    rootc_spec = (full((1, 128)) if rootc_bcast
                  else pl.BlockSpec((GBLK, 128), lambda i: (i, 0)))
    return pl.pallas_call(
        body,
        grid=(2,),
        in_specs=[pl.BlockSpec((GBLK, 128), lambda i: (i, 0)),
                  pl.BlockSpec((GBLK, 128), lambda i: (i, 0)),
                  rootc_spec,
                  pl.BlockSpec((GBLK, 128), lambda i: (i, 0)),
                  full((1, 128)), full((128, 128)), full((128, 128))],
        out_specs=[pl.BlockSpec((GBLK, 128), lambda i: (i, 0))] * 4,
        out_shape=[jax.ShapeDtypeStruct((PR, 128), jnp.float32)] * 4,
    )


@functools.lru_cache(maxsize=None)
def _tc_stage3():
    def body(a0, b0, rootc, rootv, brel, out):
        i = pl.program_id(0)
        ri = lax.broadcasted_iota(jnp.int32, (GBLK, 1), 0)
        mask = (ri < (N_NODES // 16)).astype(jnp.float32)
        consr = jax.nn.relu(a0[...] + brel[...] + rootc[...]) * mask
        varr = jax.nn.relu(b0[...] + brel[...] + rootv[...]) * mask
        part = jnp.concatenate([jnp.sum(consr, 0, keepdims=True),
                                jnp.sum(varr, 0, keepdims=True)], axis=0)
        sel = (i == 0).astype(jnp.float32)
        part4 = jnp.concatenate([part * sel, part * (1.0 - sel)], axis=0)

        @pl.when(i == 0)
        def _zero():
            out[...] = jnp.zeros_like(out)

        out[...] += part4

    def full(shape):
        return pl.BlockSpec(shape, lambda i: (0, 0))

    return pl.pallas_call(
        body,
        grid=(2,),
        in_specs=[pl.BlockSpec((GBLK, 128), lambda i: (i, 0)),
                  pl.BlockSpec((GBLK, 128), lambda i: (i, 0)),
                  pl.BlockSpec((GBLK, 128), lambda i: (i, 0)),
                  pl.BlockSpec((GBLK, 128), lambda i: (i, 0)),
                  full((1, 128))],
        out_specs=full((4, 128)),
        out_shape=jax.ShapeDtypeStruct((4, 128), jnp.float32),
    )


def _ln_row(x, g, b, eps=1e-5):
    m = jnp.mean(x, -1, keepdims=True)
    v = jnp.var(x, -1, keepdims=True)
    return (x - m) / jnp.sqrt(v + eps) * g + b


def kernel(constraint_features_s, edge_index_s, edge_attr_s,
           variable_features_s, bounds_s,
           constraint_features_t, edge_index_t, edge_attr_t,
           variable_features_t, bounds_t, params):
    p = params
    relu = jax.nn.relu
    n_edges = edge_index_s.shape[1]
    rows_pg = n_edges // EB

    c = p['ln_edge_b'][0]
    cons0row = relu(p['ln_cons_b'][0] * p['W_cons'][:, 0] + p['b_cons'])
    rootc1 = cons0row @ p['Wroot1'].T
    tabC1row = c * (cons0row @ p['Wrel1'].T)
    eye16 = jnp.eye(16, dtype=jnp.float32)

    def kron16(w):
        return jnp.kron(eye16, w.astype(jnp.float32))

    def tile16(row):
        return jnp.tile(row.reshape(1, -1), (1, 16)).reshape(1, -1)

    G6 = kron16(jnp.ones((8, 8), jnp.float32) / 6.0)
    lmask = tile16(jnp.array([1, 1, 1, 1, 1, 1, 0, 0], jnp.float32))
    pad2 = lambda r: jnp.concatenate([r, jnp.zeros((2,), jnp.float32)])
    gP = tile16(pad2(p['ln_var_g']))
    bP = tile16(pad2(p['ln_var_b']))
    WvT8 = jnp.concatenate([p['W_var'].T,
                            jnp.zeros((2, 32), jnp.float32)])
    WvK = kron16(WvT8)
    bvK = tile16(p['b_var'])
    Wroot1K = kron16(p['Wroot1'].T)
    cWrel1K = kron16(c * p['Wrel1'].T)
    brel1P = tile16(p['brel1'])
    rootc1P = tile16(rootc1)
    def pad8x8(w):
        return jnp.zeros((8, 8), jnp.float32).at[:w.shape[0], :w.shape[1]].set(w)

    def padrow8(r):
        return jnp.concatenate([r, jnp.zeros((8 - r.shape[0],), jnp.float32)])

    Wroot2K = kron16(pad8x8(p['Wroot2'].T))
    cWrel2K = kron16(pad8x8(c * p['Wrel2'].T))
    brel2P = tile16(padrow8(p['brel2']))
    Wroot3K = kron16(pad8x8(p['Wroot3'].T))
    cWrel3K = kron16(pad8x8(c * p['Wrel3'].T))
    brel3P = tile16(padrow8(p['brel3']))
    zeros8 = jnp.zeros((N_PAD, 8), jnp.float32)
    tabC1 = jnp.tile(tile16(tabC1row), (EB // 16, 1)).reshape(EB, 8)

    stage0 = _tc_stage0()
    mid1 = _tc_stage_mid(True)
    mid2 = _tc_stage_mid(False)
    stage3 = _tc_stage3()
    pass8 = _sc_pass(8, n_edges)
    pass8c = _sc_pass(8, n_edges, True)

    ei_s = edge_index_s.astype(jnp.int32).reshape(2, rows_pg, EB)
    ei_t = edge_index_t.astype(jnp.int32).reshape(2, rows_pg, EB)

    def pack_vf(vf):
        return jnp.pad(vf, ((0, N_PAD - vf.shape[0]), (0, 2)))

    vfp = jnp.concatenate([pack_vf(variable_features_s),
                           pack_vf(variable_features_t)]).reshape(PR, 128)

    rootv1, tabV1 = stage0(vfp, G6, lmask, gP, bP, WvK, bvK,
                           Wroot1K, cWrel1K)
    aggA, aggB = pass8c(ei_s, ei_t, tabV1.reshape(2 * N_PAD, 8), tabC1,
                        zeros8)
    rootc2, rootv2, tabC2, tabV2 = mid1(
        aggA.reshape(PR, 128), aggB.reshape(PR, 128), rootc1P, rootv1,
        brel1P, Wroot2K, cWrel2K)
    aggA, aggB = pass8(ei_s, ei_t, tabV2.reshape(2 * N_PAD, 8),
                       tabC2.reshape(2 * N_PAD, 8), zeros8)
    rootc3, rootv3, tabC3, tabV3 = mid2(
        aggA.reshape(PR, 128), aggB.reshape(PR, 128), rootc2, rootv2,
        brel2P, Wroot3K, cWrel3K)
    aggA, aggB = pass8(ei_s, ei_t, tabV3.reshape(2 * N_PAD, 8),
                       tabC3.reshape(2 * N_PAD, 8), zeros8)
    sums = stage3(aggA.reshape(PR, 128), aggB.reshape(PR, 128),
                  rootc3, rootv3, brel3P)
    sums = jnp.sum(sums.reshape(4, 16, 8), axis=1)[:, :4] / N_NODES

    def bnd_row(bounds):
        return relu(_ln_row(bounds, p['ln_bnd_g'], p['ln_bnd_b'])
                    @ p['W_bnd'].T + p['b_bnd'])

    out0 = jnp.concatenate([sums[1:2], sums[0:1], bnd_row(bounds_s)], axis=1)
    out1 = jnp.concatenate([sums[3:4], sums[2:3], bnd_row(bounds_t)], axis=1)
    score0 = jnp.linalg.norm(out0, axis=1)
    score1 = jnp.linalg.norm(out1, axis=1)
    return jax.nn.sigmoid(-score0 + score1)
